# baseline (speedup 1.0000x reference)
import functools

import jax
import jax.numpy as jnp
from jax import lax
from jax.experimental import pallas as pl
from jax.experimental.pallas import tpu as pltpu
from jax.experimental.pallas import tpu_sc as plsc

N_ORDER = 50000
N_DEVICE = 10000
N_TYPE = 64
H = 64

NC = 2
NS = 16
CH = 128
CL = 8
R = 4

HALF_O = N_ORDER // NC
ACC_O = 25088
ACC_D = 10112


def _elu(x):
    return jnp.where(x > 0, x, jnp.exp(jnp.minimum(x, 0.0)) - 1.0)


def _proj_body(xo, xd, xt, W_po, b_po, W_pd, b_pd, W_pt, b_pt,
               ho, hd, ht):
    ho[...] = _elu(jnp.dot(xo[...], W_po[...],
                           preferred_element_type=jnp.float32) + b_po[...])
    hd[...] = _elu(jnp.dot(xd[...], W_pd[...],
                           preferred_element_type=jnp.float32) + b_pd[...])
    ht[...] = _elu(xt[...] * W_pt[...] + b_pt[...])


def _project(xo, xd, xt, W_po, b_po, W_pd, b_pd, W_pt, b_pt):
    grid = 10
    bo, bd = N_ORDER // grid, N_DEVICE // grid
    return pl.pallas_call(
        _proj_body,
        grid=(grid,),
        in_specs=[
            pl.BlockSpec((bo, 5), lambda i: (i, 0)),
            pl.BlockSpec((bd, 6), lambda i: (i, 0)),
            pl.BlockSpec((N_TYPE, 1), lambda i: (0, 0)),
            pl.BlockSpec((5, H), lambda i: (0, 0)),
            pl.BlockSpec((H,), lambda i: (0,)),
            pl.BlockSpec((6, H), lambda i: (0, 0)),
            pl.BlockSpec((H,), lambda i: (0,)),
            pl.BlockSpec((1, H), lambda i: (0, 0)),
            pl.BlockSpec((H,), lambda i: (0,)),
        ],
        out_specs=[
            pl.BlockSpec((bo, H), lambda i: (i, 0)),
            pl.BlockSpec((bd, H), lambda i: (i, 0)),
            pl.BlockSpec((N_TYPE, H), lambda i: (0, 0)),
        ],
        out_shape=[
            jax.ShapeDtypeStruct((N_ORDER, H), jnp.float32),
            jax.ShapeDtypeStruct((N_DEVICE, H), jnp.float32),
            jax.ShapeDtypeStruct((N_TYPE, H), jnp.float32),
        ],
    )(xo, xd, xt, W_po, b_po, W_pd, b_pd, W_pt, b_pt)


ACC_CNT_O = 50176


def _pad_edges(src, dst, dst_fill):
    E = src.shape[0]
    mult = NC * NS * R * CH
    Ep = ((E + mult - 1) // mult) * mult
    if Ep != E:
        pad = Ep - E
        src = jnp.concatenate([src, jnp.zeros((pad,), jnp.int32)])
        dst = jnp.concatenate([dst, jnp.full((pad,), dst_fill, jnp.int32)])
    return src, dst


def _scan_sum(rr, cpt, chunk0, base, remap, tab, s2, d2,
              idx_s, idx_d, rbuf, gsem, ssem, acc, cnt, ones_v):
    n_super = cpt // rr
    W = rr * CH

    def super_body(g, carry):
        e0 = (chunk0 + g * rr) * CH
        pltpu.sync_copy(s2.at[pl.ds(e0, W)], idx_s)
        pltpu.sync_copy(d2.at[pl.ds(e0, W)], idx_d)
        gd = pltpu.async_copy(tab.at[idx_s], rbuf, gsem)
        if remap:
            for i in range(W // 16):
                d = idx_d[pl.ds(i * 16, 16)] - base
                ok = (d >= 0) & (d < HALF_O)
                idx_d[pl.ds(i * 16, 16)] = jnp.where(ok, d, HALF_O)
        gd.wait()
        sd = [pltpu.async_copy(rbuf, acc.at[idx_d], ssem, add=True)]
        if cnt is not None:
            sd.append(pltpu.async_copy(ones_v, cnt.at[idx_d], ssem,
                                       add=True))
        for d in sd:
            d.wait()
        return carry

    lax.fori_loop(0, n_super, super_body, 0)


def _scan_cnt(rr, cpt, chunk0, d2, idx_d, ssem, cnt, ones_v):
    n_super = cpt // rr
    W = rr * CH

    def super_body(g, carry):
        e0 = (chunk0 + g * rr) * CH
        pltpu.sync_copy(d2.at[pl.ds(e0, W)], idx_d)
        pltpu.async_copy(ones_v, cnt.at[idx_d], ssem, add=True).wait()
        return carry

    lax.fori_loop(0, n_super, super_body, 0)


def _order_sc(hd, ht, e_d2o, e_t2o):
    rr = 2
    cpt = [e[0].shape[0] // CH // NS for e in (e_d2o, e_t2o)]
    rpt = ACC_O // NS
    z64 = jnp.zeros((rpt, H), jnp.float32)

    def body(hd_t, sd2o, dd2o, ht_t, st2o, dt2o, z64_t,
             out_d2o, out_t2o,
             idx_s, idx_d, rbuf, acc, gsem, ssem):
        c = lax.axis_index("c")
        s = lax.axis_index("s")
        base = c * HALF_O
        phases = [(hd_t, sd2o, dd2o, cpt[0], out_d2o),
                  (ht_t, st2o, dt2o, cpt[1], out_t2o)]
        pltpu.sync_copy(z64_t.at[pl.ds(0, rpt)],
                        acc.at[pl.ds(s * rpt, rpt)])
        plsc.subcore_barrier()
        for pi, (tab, s2, d2, cp, out) in enumerate(phases):
            _scan_sum(rr, cp, s * cp, base, True, tab, s2, d2,
                      idx_s, idx_d, rbuf, gsem, ssem, acc, None, None)
            plsc.subcore_barrier()
            pltpu.sync_copy(acc.at[pl.ds(s * rpt, rpt)],
                            out.at[c, pl.ds(s * rpt, rpt)])
            if pi + 1 < len(phases):
                pltpu.sync_copy(z64_t.at[pl.ds(0, rpt)],
                                acc.at[pl.ds(s * rpt, rpt)])
                plsc.subcore_barrier()

    mesh = plsc.VectorSubcoreMesh(core_axis_name="c", subcore_axis_name="s")
    f = pl.kernel(
        body,
        out_type=[jax.ShapeDtypeStruct((NC, ACC_O, H), jnp.float32)] * 2,
        mesh=mesh,
        scratch_types=(
            [pltpu.VMEM((rr * CH,), jnp.int32)] * 2
            + [pltpu.VMEM((rr * CH, H), jnp.float32)]
            + [pltpu.VMEM_SHARED((ACC_O, H), jnp.float32)]
            + [pltpu.SemaphoreType.DMA] * 2),
        compiler_params=pltpu.CompilerParams(use_tc_tiling_on_sc=False),
    )
    return f(hd, e_d2o[0], e_d2o[1], ht, e_t2o[0], e_t2o[1], z64)


def _device_sc(ho, hd, ht, e_o2d, e_d2d, e_t2d, dd2o, dt2o):
    rr = R
    cpt = [e[0].shape[0] // CH // (NC * NS) for e in (e_o2d, e_d2d, e_t2d)]
    cpt_co = [d.shape[0] // CH // (NC * NS) for d in (dd2o, dt2o)]
    rptd = ACC_D // NS
    rpto = ACC_CNT_O // NS
    z64 = jnp.zeros((rptd, H), jnp.float32)
    z8 = jnp.zeros((rpto, CL), jnp.float32)
    ones_hbm = jnp.ones((R * CH, CL), jnp.float32)

    def body(ho_t, so2d, do2d, hd_t, sd2d, dd2d, ht_t, st2d, dt2d,
             dd2o_t, dt2o_t, z64_t, z8_t, ones_t,
             o_so2d, o_co2d, o_sd2d, o_cd2d, o_st2d, o_ct2d,
             o_cd2o, o_ct2o,
             idx_s, idx_d, rbuf, ones_v, acc, cnt, cnt_o, gsem, ssem):
        c = lax.axis_index("c")
        s = lax.axis_index("s")
        tile = s * NC + c
        pltpu.sync_copy(z64_t.at[pl.ds(0, rptd)],
                        acc.at[pl.ds(s * rptd, rptd)])
        pltpu.sync_copy(z8_t.at[pl.ds(0, rptd)],
                        cnt.at[pl.ds(s * rptd, rptd)])
        pltpu.sync_copy(z8_t.at[pl.ds(0, rpto)],
                        cnt_o.at[pl.ds(s * rpto, rpto)])
        pltpu.sync_copy(ones_t, ones_v)
        plsc.subcore_barrier()

        phases = [(ho_t, so2d, do2d, cpt[0], o_so2d, o_co2d),
                  (hd_t, sd2d, dd2d, cpt[1], o_sd2d, o_cd2d),
                  (ht_t, st2d, dt2d, cpt[2], o_st2d, o_ct2d)]
        for pi, (tab, s2, d2, cp, out_s, out_c) in enumerate(phases):
            _scan_sum(rr, cp, tile * cp, 0, False, tab, s2, d2,
                      idx_s, idx_d, rbuf, gsem, ssem, acc, cnt, ones_v)
            plsc.subcore_barrier()
            pltpu.sync_copy(acc.at[pl.ds(s * rptd, rptd)],
                            out_s.at[c, pl.ds(s * rptd, rptd)])
            pltpu.sync_copy(cnt.at[pl.ds(s * rptd, rptd)],
                            out_c.at[c, pl.ds(s * rptd, rptd)])
            if pi + 1 < len(phases):
                pltpu.sync_copy(z64_t.at[pl.ds(0, rptd)],
                                acc.at[pl.ds(s * rptd, rptd)])
                pltpu.sync_copy(z8_t.at[pl.ds(0, rptd)],
                                cnt.at[pl.ds(s * rptd, rptd)])
                plsc.subcore_barrier()

        cphases = [(dd2o_t, cpt_co[0], o_cd2o), (dt2o_t, cpt_co[1], o_ct2o)]
        for pi, (d2, cp, out_c) in enumerate(cphases):
            if pi == 0:
                plsc.subcore_barrier()
            _scan_cnt(rr, cp, tile * cp, d2, idx_d, ssem, cnt_o, ones_v)
            plsc.subcore_barrier()
            pltpu.sync_copy(cnt_o.at[pl.ds(s * rpto, rpto)],
                            out_c.at[c, pl.ds(s * rpto, rpto)])
            if pi + 1 < len(cphases):
                pltpu.sync_copy(z8_t.at[pl.ds(0, rpto)],
                                cnt_o.at[pl.ds(s * rpto, rpto)])
                plsc.subcore_barrier()

    mesh = plsc.VectorSubcoreMesh(core_axis_name="c", subcore_axis_name="s")
    f = pl.kernel(
        body,
        out_type=([jax.ShapeDtypeStruct((NC, ACC_D, H), jnp.float32),
                   jax.ShapeDtypeStruct((NC, ACC_D, CL), jnp.float32)] * 3
                  + [jax.ShapeDtypeStruct((NC, ACC_CNT_O, CL),
                                          jnp.float32)] * 2),
        mesh=mesh,
        scratch_types=(
            [pltpu.VMEM((rr * CH,), jnp.int32)] * 2
            + [pltpu.VMEM((rr * CH, H), jnp.float32)]
            + [pltpu.VMEM((rr * CH, CL), jnp.float32),
               pltpu.VMEM_SHARED((ACC_D, H), jnp.float32),
               pltpu.VMEM_SHARED((ACC_D, CL), jnp.float32),
               pltpu.VMEM_SHARED((ACC_CNT_O, CL), jnp.float32)]
            + [pltpu.SemaphoreType.DMA] * 2),
        compiler_params=pltpu.CompilerParams(use_tc_tiling_on_sc=False),
    )
    return f(ho, e_o2d[0], e_o2d[1], hd, e_d2d[0], e_d2d[1],
             ht, e_t2d[0], e_t2d[1], dd2o, dt2o, z64, z8, ones_hbm)


def _layer_norm(x, g, b, eps=1e-5):
    mu = jnp.mean(x, axis=-1, keepdims=True)
    xc = x - mu
    var = jnp.mean(xc * xc, axis=-1, keepdims=True)
    return xc * lax.rsqrt(var + eps) * g + b


def _mean2(sum_ref, cnt_ref):
    s = sum_ref[0] + sum_ref[1]
    n = cnt_ref[0][:, 0:1] + cnt_ref[1][:, 0:1]
    return s / jnp.maximum(n, 1.0)


def _order_body(ho, sum_d, cnt_d, sum_t, cnt_t,
                W1, W2, W3, b_uo, g_o, be_o, out):
    n_d = cnt_d[0][:, 0:1] + cnt_d[1][:, 0:1]
    n_t = cnt_t[0][:, 0:1] + cnt_t[1][:, 0:1]
    agg_d = sum_d[0] / jnp.maximum(n_d, 1.0)
    agg_t = sum_t[0] / jnp.maximum(n_t, 1.0)
    h = ho[...]
    z = (jnp.dot(h, W1[...], preferred_element_type=jnp.float32)
         + jnp.dot(agg_d, W2[...], preferred_element_type=jnp.float32)
         + jnp.dot(agg_t, W3[...], preferred_element_type=jnp.float32)
         + b_uo[...])
    out[...] = _layer_norm(h + _elu(z), g_o[...], be_o[...])


def _order_update(ho, sums_d, cnts_d, sums_t, cnts_t, W_uo, b_uo, g_o, be_o):
    W1, W2, W3 = W_uo[:H], W_uo[H:2 * H], W_uo[2 * H:]
    grid = 50
    bm = N_ORDER // grid
    per_half = HALF_O // bm

    def agg_spec():
        return pl.BlockSpec((1, bm, H), lambda i: (i // per_half,
                                                   i % per_half, 0))

    def cnt_spec():
        return pl.BlockSpec((NC, bm, CL), lambda i: (0, i, 0))

    return pl.pallas_call(
        _order_body,
        grid=(grid,),
        in_specs=[
            pl.BlockSpec((bm, H), lambda i: (i, 0)),
            agg_spec(), cnt_spec(), agg_spec(), cnt_spec(),
            pl.BlockSpec((H, H), lambda i: (0, 0)),
            pl.BlockSpec((H, H), lambda i: (0, 0)),
            pl.BlockSpec((H, H), lambda i: (0, 0)),
            pl.BlockSpec((H,), lambda i: (0,)),
            pl.BlockSpec((H,), lambda i: (0,)),
            pl.BlockSpec((H,), lambda i: (0,)),
        ],
        out_specs=pl.BlockSpec((bm, H), lambda i: (i, 0)),
        out_shape=jax.ShapeDtypeStruct((N_ORDER, H), jnp.float32),
    )(ho, sums_d, cnts_d, sums_t, cnts_t, W1, W2, W3, b_uo, g_o, be_o)


def _device_body(hd, sum_o, cnt_o, sum_d, cnt_d, sum_t, cnt_t,
                 V1, V2, V3, V4, b_ud, g_d, be_d, out):
    agg_o = _mean2(sum_o, cnt_o)
    agg_d = _mean2(sum_d, cnt_d)
    agg_t = _mean2(sum_t, cnt_t)
    h = hd[...]
    z = (jnp.dot(h, V1[...], preferred_element_type=jnp.float32)
         + jnp.dot(agg_o, V2[...], preferred_element_type=jnp.float32)
         + jnp.dot(agg_d, V3[...], preferred_element_type=jnp.float32)
         + jnp.dot(agg_t, V4[...], preferred_element_type=jnp.float32)
         + b_ud[...])
    out[...] = _layer_norm(h + _elu(z), g_d[...], be_d[...])


def _device_update(hd, so, co, sd, cd, st, ct, W_ud, b_ud, g_d, be_d):
    V1, V2, V3, V4 = (W_ud[:H], W_ud[H:2 * H],
                      W_ud[2 * H:3 * H], W_ud[3 * H:])
    grid = 10
    bm = N_DEVICE // grid

    def agg_spec():
        return pl.BlockSpec((NC, bm, H), lambda i: (0, i, 0))

    def cnt_spec():
        return pl.BlockSpec((NC, bm, CL), lambda i: (0, i, 0))

    return pl.pallas_call(
        _device_body,
        grid=(grid,),
        in_specs=[
            pl.BlockSpec((bm, H), lambda i: (i, 0)),
            agg_spec(), cnt_spec(), agg_spec(), cnt_spec(),
            agg_spec(), cnt_spec(),
            pl.BlockSpec((H, H), lambda i: (0, 0)),
            pl.BlockSpec((H, H), lambda i: (0, 0)),
            pl.BlockSpec((H, H), lambda i: (0, 0)),
            pl.BlockSpec((H, H), lambda i: (0, 0)),
            pl.BlockSpec((H,), lambda i: (0,)),
            pl.BlockSpec((H,), lambda i: (0,)),
            pl.BlockSpec((H,), lambda i: (0,)),
        ],
        out_specs=pl.BlockSpec((bm, H), lambda i: (i, 0)),
        out_shape=jax.ShapeDtypeStruct((N_DEVICE, H), jnp.float32),
    )(hd, so, co, sd, cd, st, ct, V1, V2, V3, V4, b_ud, g_d, be_d)


def kernel(x_order, x_device, x_type, src_d2o, dst_d2o, src_t2o, dst_t2o,
           src_o2d, dst_o2d, src_d2d, dst_d2d, src_t2d, dst_t2d,
           W_po, b_po, W_pd, b_pd, W_pt, b_pt, W_uo, b_uo, W_ud, b_ud,
           g_o, be_o, g_d, be_d):
    ho, hd, ht = _project(x_order, x_device, x_type,
                          W_po, b_po, W_pd, b_pd, W_pt, b_pt)

    e_d2o = _pad_edges(src_d2o, dst_d2o, N_ORDER)
    e_t2o = _pad_edges(src_t2o, dst_t2o, N_ORDER)
    e_o2d = _pad_edges(src_o2d, dst_o2d, N_DEVICE)
    e_d2d = _pad_edges(src_d2d, dst_d2d, N_DEVICE)
    e_t2d = _pad_edges(src_t2d, dst_t2d, N_DEVICE)

    s_d2o, s_t2o = _order_sc(hd, ht, e_d2o, e_t2o)
    (s_o2d, c_o2d, s_d2d, c_d2d, s_t2d, c_t2d,
     c_d2o, c_t2o) = _device_sc(ho, hd, ht, e_o2d, e_d2d, e_t2d,
                                e_d2o[1], e_t2o[1])

    ho_new = _order_update(ho, s_d2o, c_d2o, s_t2o, c_t2o,
                           W_uo, b_uo, g_o, be_o)
    hd_new = _device_update(hd, s_o2d, c_o2d, s_d2d, c_d2d, s_t2d, c_t2d,
                            W_ud, b_ud, g_d, be_d)
    return (ho_new, hd_new)


# speedup vs baseline: 1.0847x; 1.0847x over previous
import functools

import jax
import jax.numpy as jnp
from jax import lax
from jax.experimental import pallas as pl
from jax.experimental.pallas import tpu as pltpu
from jax.experimental.pallas import tpu_sc as plsc

N_ORDER = 50000
N_DEVICE = 10000
N_TYPE = 64
H = 64

NC = 2
NS = 16
CH = 128
CL = 8
R = 4

HALF_O = N_ORDER // NC
ACC_O = 25088
ACC_D = 10112


def _elu(x):
    return jnp.where(x > 0, x, jnp.exp(jnp.minimum(x, 0.0)) - 1.0)


def _proj_body(xo, xd, xt, W_po, b_po, W_pd, b_pd, W_pt, b_pt,
               ho, hd, ht):
    ho[...] = _elu(jnp.dot(xo[...], W_po[...],
                           preferred_element_type=jnp.float32) + b_po[...])
    hd[...] = _elu(jnp.dot(xd[...], W_pd[...],
                           preferred_element_type=jnp.float32) + b_pd[...])
    ht[...] = _elu(xt[...] * W_pt[...] + b_pt[...])


def _project(xo, xd, xt, W_po, b_po, W_pd, b_pd, W_pt, b_pt):
    grid = 10
    bo, bd = N_ORDER // grid, N_DEVICE // grid
    return pl.pallas_call(
        _proj_body,
        grid=(grid,),
        in_specs=[
            pl.BlockSpec((bo, 5), lambda i: (i, 0)),
            pl.BlockSpec((bd, 6), lambda i: (i, 0)),
            pl.BlockSpec((N_TYPE, 1), lambda i: (0, 0)),
            pl.BlockSpec((5, H), lambda i: (0, 0)),
            pl.BlockSpec((H,), lambda i: (0,)),
            pl.BlockSpec((6, H), lambda i: (0, 0)),
            pl.BlockSpec((H,), lambda i: (0,)),
            pl.BlockSpec((1, H), lambda i: (0, 0)),
            pl.BlockSpec((H,), lambda i: (0,)),
        ],
        out_specs=[
            pl.BlockSpec((bo, H), lambda i: (i, 0)),
            pl.BlockSpec((bd, H), lambda i: (i, 0)),
            pl.BlockSpec((N_TYPE, H), lambda i: (0, 0)),
        ],
        out_shape=[
            jax.ShapeDtypeStruct((N_ORDER, H), jnp.float32),
            jax.ShapeDtypeStruct((N_DEVICE, H), jnp.float32),
            jax.ShapeDtypeStruct((N_TYPE, H), jnp.float32),
        ],
    )(xo, xd, xt, W_po, b_po, W_pd, b_pd, W_pt, b_pt)


ACC_CNT_O = 50176


def _pad_edges(src, dst, dst_fill):
    E = src.shape[0]
    mult = NC * NS * R * CH
    Ep = ((E + mult - 1) // mult) * mult
    if Ep != E:
        pad = Ep - E
        src = jnp.concatenate([src, jnp.zeros((pad,), jnp.int32)])
        dst = jnp.concatenate([dst, jnp.full((pad,), dst_fill, jnp.int32)])
    return src, dst


def _scan_pipe(rr, cpt, chunk0, base, remap, tab, s2, d2,
               sets, gsems, ssems, isems, acc, cnt, ones_v):
    n_super = cpt // rr
    W = rr * CH
    assert n_super >= 2 and (n_super - 2) % 3 == 0 or n_super == 2, n_super
    T = (n_super - 2) // 3

    def issue_idx(p, sup):
        e0 = (chunk0 + sup * rr) * CH
        if tab is not None:
            pltpu.async_copy(s2.at[pl.ds(e0, W)], sets[p][0], isems[p])
        pltpu.async_copy(d2.at[pl.ds(e0, W)], sets[p][1], isems[p])

    def wait_idx(p):
        if tab is not None:
            pltpu.make_async_copy(s2.at[pl.ds(0, W)], sets[p][0],
                                  isems[p]).wait()
        pltpu.make_async_copy(d2.at[pl.ds(0, W)], sets[p][1],
                              isems[p]).wait()

    def drain_scatter(p):
        if tab is not None:
            pltpu.make_async_copy(sets[p][2], acc.at[sets[p][1]],
                                  ssems[p]).wait()
        if cnt is not None:
            pltpu.make_async_copy(ones_v, cnt.at[sets[p][1]],
                                  ssems[p]).wait()

    def stage(p):
        idx_s, idx_d, rbuf = sets[p]
        wait_idx(p)
        gd = None
        if tab is not None:
            gd = pltpu.async_copy(tab.at[idx_s], rbuf, gsems[p])
        if remap:
            for i in range(W // 16):
                d = idx_d[pl.ds(i * 16, 16)] - base
                ok = (d >= 0) & (d < HALF_O)
                idx_d[pl.ds(i * 16, 16)] = jnp.where(ok, d, HALF_O)
        if gd is not None:
            gd.wait()
            pltpu.async_copy(rbuf, acc.at[idx_d], ssems[p], add=True)
        if cnt is not None:
            pltpu.async_copy(ones_v, cnt.at[idx_d], ssems[p], add=True)

    issue_idx(0, 0)
    issue_idx(1, 1)
    stage(0)
    if n_super > 2:
        issue_idx(2, 2)
    stage(1)
    drain_scatter(0)
    if n_super > 3:
        issue_idx(0, 3)

    if T > 0:
        def body(t, carry):
            for k in range(3):
                g = 2 + 3 * t + k
                p = (2 + k) % 3
                q = (p + 2) % 3
                stage(p)
                drain_scatter(q)
                issue_idx(q, jnp.minimum(g + 2, n_super - 1))
            return carry

        lax.fori_loop(0, T, body, 0)
        wait_idx((n_super - 3) % 3)
        wait_idx((n_super - 2) % 3)
    drain_scatter((n_super - 1) % 3)


def _order_sc(hd, ht, e_d2o, e_t2o):
    rr = 1
    cpt = [e[0].shape[0] // CH // NS for e in (e_d2o, e_t2o)]
    rpt = ACC_O // NS
    z64 = jnp.zeros((rpt, H), jnp.float32)

    def body(hd_t, sd2o, dd2o, ht_t, st2o, dt2o, z64_t,
             out_d2o, out_t2o, *scr):
        sets = [scr[3 * j:3 * j + 3] for j in range(3)]
        gsems, ssems, isems = scr[9:12], scr[12:15], scr[15:18]
        c = lax.axis_index("c")
        s = lax.axis_index("s")
        base = c * HALF_O
        phases = [(hd_t, sd2o, dd2o, cpt[0], out_d2o),
                  (ht_t, st2o, dt2o, cpt[1], out_t2o)]
        acc = scr[18]
        pltpu.sync_copy(z64_t.at[pl.ds(0, rpt)],
                        acc.at[pl.ds(s * rpt, rpt)])
        plsc.subcore_barrier()
        for pi, (tab, s2, d2, cp, out) in enumerate(phases):
            _scan_pipe(rr, cp, s * cp, base, True, tab, s2, d2,
                       sets, gsems, ssems, isems, acc, None, None)
            plsc.subcore_barrier()
            pltpu.sync_copy(acc.at[pl.ds(s * rpt, rpt)],
                            out.at[c, pl.ds(s * rpt, rpt)])
            if pi + 1 < len(phases):
                pltpu.sync_copy(z64_t.at[pl.ds(0, rpt)],
                                acc.at[pl.ds(s * rpt, rpt)])
                plsc.subcore_barrier()

    mesh = plsc.VectorSubcoreMesh(core_axis_name="c", subcore_axis_name="s")
    W = rr * CH
    f = pl.kernel(
        body,
        out_type=[jax.ShapeDtypeStruct((NC, ACC_O, H), jnp.float32)] * 2,
        mesh=mesh,
        scratch_types=(
            [pltpu.VMEM((W,), jnp.int32),
             pltpu.VMEM((W,), jnp.int32),
             pltpu.VMEM((W, H), jnp.float32)] * 3
            + [pltpu.SemaphoreType.DMA] * 9
            + [pltpu.VMEM_SHARED((ACC_O, H), jnp.float32)]),
        compiler_params=pltpu.CompilerParams(use_tc_tiling_on_sc=False),
    )
    return f(hd, e_d2o[0], e_d2o[1], ht, e_t2o[0], e_t2o[1], z64)


def _device_sc(ho, hd, ht, e_o2d, e_d2d, e_t2d, dd2o, dt2o):
    rr = 2
    cpt = [e[0].shape[0] // CH // (NC * NS) for e in (e_o2d, e_d2d, e_t2d)]
    cpt_co = [d.shape[0] // CH // (NC * NS) for d in (dd2o, dt2o)]
    rptd = ACC_D // NS
    rpto = ACC_CNT_O // NS
    W = rr * CH
    z64 = jnp.zeros((rptd, H), jnp.float32)
    z8 = jnp.zeros((rpto, CL), jnp.float32)
    ones_hbm = jnp.ones((W, CL), jnp.float32)

    def body(ho_t, so2d, do2d, hd_t, sd2d, dd2d, ht_t, st2d, dt2d,
             dd2o_t, dt2o_t, z64_t, z8_t, ones_t,
             o_so2d, o_co2d, o_sd2d, o_cd2d, o_st2d, o_ct2d,
             o_cd2o, o_ct2o, *scr):
        sets = [scr[3 * j:3 * j + 3] for j in range(3)]
        ones_v = scr[9]
        gsems, ssems, isems = scr[10:13], scr[13:16], scr[16:19]
        acc, cnt, cnt_o = scr[19], scr[20], scr[21]
        c = lax.axis_index("c")
        s = lax.axis_index("s")
        tile = s * NC + c
        pltpu.sync_copy(z64_t.at[pl.ds(0, rptd)],
                        acc.at[pl.ds(s * rptd, rptd)])
        pltpu.sync_copy(z8_t.at[pl.ds(0, rptd)],
                        cnt.at[pl.ds(s * rptd, rptd)])
        pltpu.sync_copy(z8_t.at[pl.ds(0, rpto)],
                        cnt_o.at[pl.ds(s * rpto, rpto)])
        pltpu.sync_copy(ones_t, ones_v)
        plsc.subcore_barrier()

        phases = [(ho_t, so2d, do2d, cpt[0], o_so2d, o_co2d),
                  (hd_t, sd2d, dd2d, cpt[1], o_sd2d, o_cd2d),
                  (ht_t, st2d, dt2d, cpt[2], o_st2d, o_ct2d)]
        for pi, (tab, s2, d2, cp, out_s, out_c) in enumerate(phases):
            _scan_pipe(rr, cp, tile * cp, 0, False, tab, s2, d2,
                       sets, gsems, ssems, isems, acc, cnt, ones_v)
            plsc.subcore_barrier()
            pltpu.sync_copy(acc.at[pl.ds(s * rptd, rptd)],
                            out_s.at[c, pl.ds(s * rptd, rptd)])
            pltpu.sync_copy(cnt.at[pl.ds(s * rptd, rptd)],
                            out_c.at[c, pl.ds(s * rptd, rptd)])
            if pi + 1 < len(phases):
                pltpu.sync_copy(z64_t.at[pl.ds(0, rptd)],
                                acc.at[pl.ds(s * rptd, rptd)])
                pltpu.sync_copy(z8_t.at[pl.ds(0, rptd)],
                                cnt.at[pl.ds(s * rptd, rptd)])
                plsc.subcore_barrier()

        cphases = [(dd2o_t, cpt_co[0], o_cd2o), (dt2o_t, cpt_co[1], o_ct2o)]
        for pi, (d2, cp, out_c) in enumerate(cphases):
            if pi == 0:
                plsc.subcore_barrier()
            _scan_pipe(rr, cp, tile * cp, 0, False, None, None, d2,
                       sets, gsems, ssems, isems, None, cnt_o, ones_v)
            plsc.subcore_barrier()
            pltpu.sync_copy(cnt_o.at[pl.ds(s * rpto, rpto)],
                            out_c.at[c, pl.ds(s * rpto, rpto)])
            if pi + 1 < len(cphases):
                pltpu.sync_copy(z8_t.at[pl.ds(0, rpto)],
                                cnt_o.at[pl.ds(s * rpto, rpto)])
                plsc.subcore_barrier()

    mesh = plsc.VectorSubcoreMesh(core_axis_name="c", subcore_axis_name="s")
    f = pl.kernel(
        body,
        out_type=([jax.ShapeDtypeStruct((NC, ACC_D, H), jnp.float32),
                   jax.ShapeDtypeStruct((NC, ACC_D, CL), jnp.float32)] * 3
                  + [jax.ShapeDtypeStruct((NC, ACC_CNT_O, CL),
                                          jnp.float32)] * 2),
        mesh=mesh,
        scratch_types=(
            [pltpu.VMEM((W,), jnp.int32),
             pltpu.VMEM((W,), jnp.int32),
             pltpu.VMEM((W, H), jnp.float32)] * 3
            + [pltpu.VMEM((W, CL), jnp.float32)]
            + [pltpu.SemaphoreType.DMA] * 9
            + [pltpu.VMEM_SHARED((ACC_D, H), jnp.float32),
               pltpu.VMEM_SHARED((ACC_D, CL), jnp.float32),
               pltpu.VMEM_SHARED((ACC_CNT_O, CL), jnp.float32)]),
        compiler_params=pltpu.CompilerParams(use_tc_tiling_on_sc=False),
    )
    return f(ho, e_o2d[0], e_o2d[1], hd, e_d2d[0], e_d2d[1],
             ht, e_t2d[0], e_t2d[1], dd2o, dt2o, z64, z8, ones_hbm)


def _layer_norm(x, g, b, eps=1e-5):
    mu = jnp.mean(x, axis=-1, keepdims=True)
    xc = x - mu
    var = jnp.mean(xc * xc, axis=-1, keepdims=True)
    return xc * lax.rsqrt(var + eps) * g + b


def _mean2(sum_ref, cnt_ref):
    s = sum_ref[0] + sum_ref[1]
    n = cnt_ref[0][:, 0:1] + cnt_ref[1][:, 0:1]
    return s / jnp.maximum(n, 1.0)


def _order_body(ho, sum_d, cnt_d, sum_t, cnt_t,
                W1, W2, W3, b_uo, g_o, be_o, out):
    n_d = cnt_d[0][:, 0:1] + cnt_d[1][:, 0:1]
    n_t = cnt_t[0][:, 0:1] + cnt_t[1][:, 0:1]
    agg_d = sum_d[0] / jnp.maximum(n_d, 1.0)
    agg_t = sum_t[0] / jnp.maximum(n_t, 1.0)
    h = ho[...]
    z = (jnp.dot(h, W1[...], preferred_element_type=jnp.float32)
         + jnp.dot(agg_d, W2[...], preferred_element_type=jnp.float32)
         + jnp.dot(agg_t, W3[...], preferred_element_type=jnp.float32)
         + b_uo[...])
    out[...] = _layer_norm(h + _elu(z), g_o[...], be_o[...])


def _order_update(ho, sums_d, cnts_d, sums_t, cnts_t, W_uo, b_uo, g_o, be_o):
    W1, W2, W3 = W_uo[:H], W_uo[H:2 * H], W_uo[2 * H:]
    grid = 50
    bm = N_ORDER // grid
    per_half = HALF_O // bm

    def agg_spec():
        return pl.BlockSpec((1, bm, H), lambda i: (i // per_half,
                                                   i % per_half, 0))

    def cnt_spec():
        return pl.BlockSpec((NC, bm, CL), lambda i: (0, i, 0))

    return pl.pallas_call(
        _order_body,
        grid=(grid,),
        in_specs=[
            pl.BlockSpec((bm, H), lambda i: (i, 0)),
            agg_spec(), cnt_spec(), agg_spec(), cnt_spec(),
            pl.BlockSpec((H, H), lambda i: (0, 0)),
            pl.BlockSpec((H, H), lambda i: (0, 0)),
            pl.BlockSpec((H, H), lambda i: (0, 0)),
            pl.BlockSpec((H,), lambda i: (0,)),
            pl.BlockSpec((H,), lambda i: (0,)),
            pl.BlockSpec((H,), lambda i: (0,)),
        ],
        out_specs=pl.BlockSpec((bm, H), lambda i: (i, 0)),
        out_shape=jax.ShapeDtypeStruct((N_ORDER, H), jnp.float32),
    )(ho, sums_d, cnts_d, sums_t, cnts_t, W1, W2, W3, b_uo, g_o, be_o)


def _device_body(hd, sum_o, cnt_o, sum_d, cnt_d, sum_t, cnt_t,
                 V1, V2, V3, V4, b_ud, g_d, be_d, out):
    agg_o = _mean2(sum_o, cnt_o)
    agg_d = _mean2(sum_d, cnt_d)
    agg_t = _mean2(sum_t, cnt_t)
    h = hd[...]
    z = (jnp.dot(h, V1[...], preferred_element_type=jnp.float32)
         + jnp.dot(agg_o, V2[...], preferred_element_type=jnp.float32)
         + jnp.dot(agg_d, V3[...], preferred_element_type=jnp.float32)
         + jnp.dot(agg_t, V4[...], preferred_element_type=jnp.float32)
         + b_ud[...])
    out[...] = _layer_norm(h + _elu(z), g_d[...], be_d[...])


def _device_update(hd, so, co, sd, cd, st, ct, W_ud, b_ud, g_d, be_d):
    V1, V2, V3, V4 = (W_ud[:H], W_ud[H:2 * H],
                      W_ud[2 * H:3 * H], W_ud[3 * H:])
    grid = 10
    bm = N_DEVICE // grid

    def agg_spec():
        return pl.BlockSpec((NC, bm, H), lambda i: (0, i, 0))

    def cnt_spec():
        return pl.BlockSpec((NC, bm, CL), lambda i: (0, i, 0))

    return pl.pallas_call(
        _device_body,
        grid=(grid,),
        in_specs=[
            pl.BlockSpec((bm, H), lambda i: (i, 0)),
            agg_spec(), cnt_spec(), agg_spec(), cnt_spec(),
            agg_spec(), cnt_spec(),
            pl.BlockSpec((H, H), lambda i: (0, 0)),
            pl.BlockSpec((H, H), lambda i: (0, 0)),
            pl.BlockSpec((H, H), lambda i: (0, 0)),
            pl.BlockSpec((H, H), lambda i: (0, 0)),
            pl.BlockSpec((H,), lambda i: (0,)),
            pl.BlockSpec((H,), lambda i: (0,)),
            pl.BlockSpec((H,), lambda i: (0,)),
        ],
        out_specs=pl.BlockSpec((bm, H), lambda i: (i, 0)),
        out_shape=jax.ShapeDtypeStruct((N_DEVICE, H), jnp.float32),
    )(hd, so, co, sd, cd, st, ct, V1, V2, V3, V4, b_ud, g_d, be_d)


def kernel(x_order, x_device, x_type, src_d2o, dst_d2o, src_t2o, dst_t2o,
           src_o2d, dst_o2d, src_d2d, dst_d2d, src_t2d, dst_t2d,
           W_po, b_po, W_pd, b_pd, W_pt, b_pt, W_uo, b_uo, W_ud, b_ud,
           g_o, be_o, g_d, be_d):
    ho, hd, ht = _project(x_order, x_device, x_type,
                          W_po, b_po, W_pd, b_pd, W_pt, b_pt)

    e_d2o = _pad_edges(src_d2o, dst_d2o, N_ORDER)
    e_t2o = _pad_edges(src_t2o, dst_t2o, N_ORDER)
    e_o2d = _pad_edges(src_o2d, dst_o2d, N_DEVICE)
    e_d2d = _pad_edges(src_d2d, dst_d2d, N_DEVICE)
    e_t2d = _pad_edges(src_t2d, dst_t2d, N_DEVICE)

    s_d2o, s_t2o = _order_sc(hd, ht, e_d2o, e_t2o)
    (s_o2d, c_o2d, s_d2d, c_d2d, s_t2d, c_t2d,
     c_d2o, c_t2o) = _device_sc(ho, hd, ht, e_o2d, e_d2d, e_t2d,
                                e_d2o[1], e_t2o[1])

    ho_new = _order_update(ho, s_d2o, c_d2o, s_t2o, c_t2o,
                           W_uo, b_uo, g_o, be_o)
    hd_new = _device_update(hd, s_o2d, c_o2d, s_d2d, c_d2d, s_t2d, c_t2d,
                            W_ud, b_ud, g_d, be_d)
    return (ho_new, hd_new)


# speedup vs baseline: 1.5965x; 1.4718x over previous
import functools

import jax
import jax.numpy as jnp
from jax import lax
from jax.experimental import pallas as pl
from jax.experimental.pallas import tpu as pltpu
from jax.experimental.pallas import tpu_sc as plsc

N_ORDER = 50000
N_DEVICE = 10000
N_TYPE = 64
H = 64

NC = 2
NS = 16
CH = 128
CL = 8
R = 4

HALF_O = N_ORDER // NC
ACC_O = 25088
ACC_D = 10112


def _elu(x):
    return jnp.where(x > 0, x, jnp.exp(jnp.minimum(x, 0.0)) - 1.0)


def _proj_body(xo, xd, xt, W_po, b_po, W_pd, b_pd, W_pt, b_pt,
               ho, hd, ht):
    ho[...] = _elu(jnp.dot(xo[...], W_po[...],
                           preferred_element_type=jnp.float32) + b_po[...])
    hd[...] = _elu(jnp.dot(xd[...], W_pd[...],
                           preferred_element_type=jnp.float32) + b_pd[...])
    ht[...] = _elu(xt[...] * W_pt[...] + b_pt[...])


def _project(xo, xd, xt, W_po, b_po, W_pd, b_pd, W_pt, b_pt):
    grid = 10
    bo, bd = N_ORDER // grid, N_DEVICE // grid
    return pl.pallas_call(
        _proj_body,
        grid=(grid,),
        in_specs=[
            pl.BlockSpec((bo, 5), lambda i: (i, 0)),
            pl.BlockSpec((bd, 6), lambda i: (i, 0)),
            pl.BlockSpec((N_TYPE, 1), lambda i: (0, 0)),
            pl.BlockSpec((5, H), lambda i: (0, 0)),
            pl.BlockSpec((H,), lambda i: (0,)),
            pl.BlockSpec((6, H), lambda i: (0, 0)),
            pl.BlockSpec((H,), lambda i: (0,)),
            pl.BlockSpec((1, H), lambda i: (0, 0)),
            pl.BlockSpec((H,), lambda i: (0,)),
        ],
        out_specs=[
            pl.BlockSpec((bo, H), lambda i: (i, 0)),
            pl.BlockSpec((bd, H), lambda i: (i, 0)),
            pl.BlockSpec((N_TYPE, H), lambda i: (0, 0)),
        ],
        out_shape=[
            jax.ShapeDtypeStruct((N_ORDER, H), jnp.float32),
            jax.ShapeDtypeStruct((N_DEVICE, H), jnp.float32),
            jax.ShapeDtypeStruct((N_TYPE, H), jnp.float32),
        ],
    )(xo, xd, xt, W_po, b_po, W_pd, b_pd, W_pt, b_pt)


ACC_CNT_O = 50176


def _pad_edges(src, dst, dst_fill):
    E = src.shape[0]
    mult = NC * NS * R * CH
    Ep = ((E + mult - 1) // mult) * mult
    if Ep != E:
        pad = Ep - E
        src = jnp.concatenate([src, jnp.zeros((pad,), jnp.int32)])
        dst = jnp.concatenate([dst, jnp.full((pad,), dst_fill, jnp.int32)])
    return src, dst


def _scan_pipe(rr, cpt, chunk0, base, remap, tab, s2, d2,
               sets, gsems, ssems, isems, acc, cnt, ones_v):
    n_super = cpt // rr
    W = rr * CH
    assert n_super >= 2 and (n_super - 2) % 3 == 0 or n_super == 2, n_super
    T = (n_super - 2) // 3

    def issue_idx(p, sup):
        e0 = (chunk0 + sup * rr) * CH
        if tab is not None:
            pltpu.async_copy(s2.at[pl.ds(e0, W)], sets[p][0], isems[p])
        pltpu.async_copy(d2.at[pl.ds(e0, W)], sets[p][1], isems[p])

    def wait_idx(p):
        if tab is not None:
            pltpu.make_async_copy(s2.at[pl.ds(0, W)], sets[p][0],
                                  isems[p]).wait()
        pltpu.make_async_copy(d2.at[pl.ds(0, W)], sets[p][1],
                              isems[p]).wait()

    def drain_scatter(p):
        if tab is not None:
            pltpu.make_async_copy(sets[p][2], acc.at[sets[p][1]],
                                  ssems[p]).wait()
        if cnt is not None:
            pltpu.make_async_copy(ones_v, cnt.at[sets[p][1]],
                                  ssems[p]).wait()

    def stage(p):
        idx_s, idx_d, rbuf = sets[p]
        wait_idx(p)
        gd = None
        if tab is not None:
            gd = pltpu.async_copy(tab.at[idx_s], rbuf, gsems[p])
        if remap:
            for i in range(W // 16):
                d = idx_d[pl.ds(i * 16, 16)] - base
                ok = (d >= 0) & (d < HALF_O)
                idx_d[pl.ds(i * 16, 16)] = jnp.where(ok, d, HALF_O)
        if gd is not None:
            gd.wait()
            pltpu.async_copy(rbuf, acc.at[idx_d], ssems[p], add=True)
        if cnt is not None:
            pltpu.async_copy(ones_v, cnt.at[idx_d], ssems[p], add=True)

    issue_idx(0, 0)
    issue_idx(1, 1)
    stage(0)
    if n_super > 2:
        issue_idx(2, 2)
    stage(1)
    drain_scatter(0)
    if n_super > 3:
        issue_idx(0, 3)

    if T > 0:
        def body(t, carry):
            for k in range(3):
                g = 2 + 3 * t + k
                p = (2 + k) % 3
                q = (p + 2) % 3
                stage(p)
                drain_scatter(q)
                issue_idx(q, jnp.minimum(g + 2, n_super - 1))
            return carry

        lax.fori_loop(0, T, body, 0)
        wait_idx((n_super - 3) % 3)
        wait_idx((n_super - 2) % 3)
    drain_scatter((n_super - 1) % 3)


HW = H // NC
ACC_FULL_O = 50176


def _order_sc(hd2, ht2, e_d2o, e_t2o):
    rr = 1
    cpt = [e[0].shape[0] // CH // NS for e in (e_d2o, e_t2o)]
    rpt = ACC_FULL_O // NS
    z32 = jnp.zeros((rpt, HW), jnp.float32)

    def body(hd_t, sd2o, dd2o, ht_t, st2o, dt2o, z32_t,
             out_d2o, out_t2o, *scr):
        sets = [scr[3 * j:3 * j + 3] for j in range(3)]
        gsems, ssems, isems = scr[9:12], scr[12:15], scr[15:18]
        c = lax.axis_index("c")
        s = lax.axis_index("s")
        phases = [(hd_t.at[c], sd2o, dd2o, cpt[0], out_d2o),
                  (ht_t.at[c], st2o, dt2o, cpt[1], out_t2o)]
        acc = scr[18]
        pltpu.sync_copy(z32_t.at[pl.ds(0, rpt)],
                        acc.at[pl.ds(s * rpt, rpt)])
        plsc.subcore_barrier()
        for pi, (tab, s2, d2, cp, out) in enumerate(phases):
            _scan_pipe(rr, cp, s * cp, 0, False, tab, s2, d2,
                       sets, gsems, ssems, isems, acc, None, None)
            plsc.subcore_barrier()
            pltpu.sync_copy(acc.at[pl.ds(s * rpt, rpt)],
                            out.at[c, pl.ds(s * rpt, rpt)])
            if pi + 1 < len(phases):
                pltpu.sync_copy(z32_t.at[pl.ds(0, rpt)],
                                acc.at[pl.ds(s * rpt, rpt)])
                plsc.subcore_barrier()

    mesh = plsc.VectorSubcoreMesh(core_axis_name="c", subcore_axis_name="s")
    W = rr * CH
    f = pl.kernel(
        body,
        out_type=[jax.ShapeDtypeStruct((NC, ACC_FULL_O, HW),
                                       jnp.float32)] * 2,
        mesh=mesh,
        scratch_types=(
            [pltpu.VMEM((W,), jnp.int32),
             pltpu.VMEM((W,), jnp.int32),
             pltpu.VMEM((W, HW), jnp.float32)] * 3
            + [pltpu.SemaphoreType.DMA] * 9
            + [pltpu.VMEM_SHARED((ACC_FULL_O, HW), jnp.float32)]),
        compiler_params=pltpu.CompilerParams(use_tc_tiling_on_sc=False),
    )
    return f(hd2, e_d2o[0], e_d2o[1], ht2, e_t2o[0], e_t2o[1], z32)


def _device_sc(ho, hd, ht, e_o2d, e_d2d, e_t2d, dd2o, dt2o):
    rr = 2
    cpt = [e[0].shape[0] // CH // (NC * NS) for e in (e_o2d, e_d2d, e_t2d)]
    cpt_co = [d.shape[0] // CH // (NC * NS) for d in (dd2o, dt2o)]
    rptd = ACC_D // NS
    rpto = ACC_CNT_O // NS
    W = rr * CH
    z64 = jnp.zeros((rptd, H), jnp.float32)
    z8 = jnp.zeros((rpto, CL), jnp.float32)
    ones_hbm = jnp.ones((W, CL), jnp.float32)

    def body(ho_t, so2d, do2d, hd_t, sd2d, dd2d, ht_t, st2d, dt2d,
             dd2o_t, dt2o_t, z64_t, z8_t, ones_t,
             o_so2d, o_co2d, o_sd2d, o_cd2d, o_st2d, o_ct2d,
             o_cd2o, o_ct2o, *scr):
        sets = [scr[3 * j:3 * j + 3] for j in range(3)]
        ones_v = scr[9]
        gsems, ssems, isems = scr[10:13], scr[13:16], scr[16:19]
        acc, cnt, cnt_o = scr[19], scr[20], scr[21]
        c = lax.axis_index("c")
        s = lax.axis_index("s")
        tile = s * NC + c
        pltpu.sync_copy(z64_t.at[pl.ds(0, rptd)],
                        acc.at[pl.ds(s * rptd, rptd)])
        pltpu.sync_copy(z8_t.at[pl.ds(0, rptd)],
                        cnt.at[pl.ds(s * rptd, rptd)])
        pltpu.sync_copy(z8_t.at[pl.ds(0, rpto)],
                        cnt_o.at[pl.ds(s * rpto, rpto)])
        pltpu.sync_copy(ones_t, ones_v)
        plsc.subcore_barrier()

        phases = [(ho_t, so2d, do2d, cpt[0], o_so2d, o_co2d),
                  (hd_t, sd2d, dd2d, cpt[1], o_sd2d, o_cd2d),
                  (ht_t, st2d, dt2d, cpt[2], o_st2d, o_ct2d)]
        for pi, (tab, s2, d2, cp, out_s, out_c) in enumerate(phases):
            _scan_pipe(rr, cp, tile * cp, 0, False, tab, s2, d2,
                       sets, gsems, ssems, isems, acc, cnt, ones_v)
            plsc.subcore_barrier()
            pltpu.sync_copy(acc.at[pl.ds(s * rptd, rptd)],
                            out_s.at[c, pl.ds(s * rptd, rptd)])
            pltpu.sync_copy(cnt.at[pl.ds(s * rptd, rptd)],
                            out_c.at[c, pl.ds(s * rptd, rptd)])
            if pi + 1 < len(phases):
                pltpu.sync_copy(z64_t.at[pl.ds(0, rptd)],
                                acc.at[pl.ds(s * rptd, rptd)])
                pltpu.sync_copy(z8_t.at[pl.ds(0, rptd)],
                                cnt.at[pl.ds(s * rptd, rptd)])
                plsc.subcore_barrier()

        cphases = [(dd2o_t, cpt_co[0], o_cd2o), (dt2o_t, cpt_co[1], o_ct2o)]
        for pi, (d2, cp, out_c) in enumerate(cphases):
            if pi == 0:
                plsc.subcore_barrier()
            _scan_pipe(rr, cp, tile * cp, 0, False, None, None, d2,
                       sets, gsems, ssems, isems, None, cnt_o, ones_v)
            plsc.subcore_barrier()
            pltpu.sync_copy(cnt_o.at[pl.ds(s * rpto, rpto)],
                            out_c.at[c, pl.ds(s * rpto, rpto)])
            if pi + 1 < len(cphases):
                pltpu.sync_copy(z8_t.at[pl.ds(0, rpto)],
                                cnt_o.at[pl.ds(s * rpto, rpto)])
                plsc.subcore_barrier()

    mesh = plsc.VectorSubcoreMesh(core_axis_name="c", subcore_axis_name="s")
    f = pl.kernel(
        body,
        out_type=([jax.ShapeDtypeStruct((NC, ACC_D, H), jnp.float32),
                   jax.ShapeDtypeStruct((NC, ACC_D, CL), jnp.float32)] * 3
                  + [jax.ShapeDtypeStruct((NC, ACC_CNT_O, CL),
                                          jnp.float32)] * 2),
        mesh=mesh,
        scratch_types=(
            [pltpu.VMEM((W,), jnp.int32),
             pltpu.VMEM((W,), jnp.int32),
             pltpu.VMEM((W, H), jnp.float32)] * 3
            + [pltpu.VMEM((W, CL), jnp.float32)]
            + [pltpu.SemaphoreType.DMA] * 9
            + [pltpu.VMEM_SHARED((ACC_D, H), jnp.float32),
               pltpu.VMEM_SHARED((ACC_D, CL), jnp.float32),
               pltpu.VMEM_SHARED((ACC_CNT_O, CL), jnp.float32)]),
        compiler_params=pltpu.CompilerParams(use_tc_tiling_on_sc=False),
    )
    return f(ho, e_o2d[0], e_o2d[1], hd, e_d2d[0], e_d2d[1],
             ht, e_t2d[0], e_t2d[1], dd2o, dt2o, z64, z8, ones_hbm)


def _layer_norm(x, g, b, eps=1e-5):
    mu = jnp.mean(x, axis=-1, keepdims=True)
    xc = x - mu
    var = jnp.mean(xc * xc, axis=-1, keepdims=True)
    return xc * lax.rsqrt(var + eps) * g + b


def _mean2(sum_ref, cnt_ref):
    s = sum_ref[0] + sum_ref[1]
    n = cnt_ref[0][:, 0:1] + cnt_ref[1][:, 0:1]
    return s / jnp.maximum(n, 1.0)


def _order_body(ho, sum_d, cnt_d, sum_t, cnt_t,
                W1, W2, W3, b_uo, g_o, be_o, out):
    n_d = cnt_d[0][:, 0:1] + cnt_d[1][:, 0:1]
    n_t = cnt_t[0][:, 0:1] + cnt_t[1][:, 0:1]
    agg_d = jnp.concatenate([sum_d[0], sum_d[1]],
                            axis=-1) / jnp.maximum(n_d, 1.0)
    agg_t = jnp.concatenate([sum_t[0], sum_t[1]],
                            axis=-1) / jnp.maximum(n_t, 1.0)
    h = ho[...]
    z = (jnp.dot(h, W1[...], preferred_element_type=jnp.float32)
         + jnp.dot(agg_d, W2[...], preferred_element_type=jnp.float32)
         + jnp.dot(agg_t, W3[...], preferred_element_type=jnp.float32)
         + b_uo[...])
    out[...] = _layer_norm(h + _elu(z), g_o[...], be_o[...])


def _order_update(ho, sums_d, cnts_d, sums_t, cnts_t, W_uo, b_uo, g_o, be_o):
    W1, W2, W3 = W_uo[:H], W_uo[H:2 * H], W_uo[2 * H:]
    grid = 50
    bm = N_ORDER // grid

    def agg_spec():
        return pl.BlockSpec((NC, bm, HW), lambda i: (0, i, 0))

    def cnt_spec():
        return pl.BlockSpec((NC, bm, CL), lambda i: (0, i, 0))

    return pl.pallas_call(
        _order_body,
        grid=(grid,),
        in_specs=[
            pl.BlockSpec((bm, H), lambda i: (i, 0)),
            agg_spec(), cnt_spec(), agg_spec(), cnt_spec(),
            pl.BlockSpec((H, H), lambda i: (0, 0)),
            pl.BlockSpec((H, H), lambda i: (0, 0)),
            pl.BlockSpec((H, H), lambda i: (0, 0)),
            pl.BlockSpec((H,), lambda i: (0,)),
            pl.BlockSpec((H,), lambda i: (0,)),
            pl.BlockSpec((H,), lambda i: (0,)),
        ],
        out_specs=pl.BlockSpec((bm, H), lambda i: (i, 0)),
        out_shape=jax.ShapeDtypeStruct((N_ORDER, H), jnp.float32),
    )(ho, sums_d, cnts_d, sums_t, cnts_t, W1, W2, W3, b_uo, g_o, be_o)


def _device_body(hd, sum_o, cnt_o, sum_d, cnt_d, sum_t, cnt_t,
                 V1, V2, V3, V4, b_ud, g_d, be_d, out):
    agg_o = _mean2(sum_o, cnt_o)
    agg_d = _mean2(sum_d, cnt_d)
    agg_t = _mean2(sum_t, cnt_t)
    h = hd[...]
    z = (jnp.dot(h, V1[...], preferred_element_type=jnp.float32)
         + jnp.dot(agg_o, V2[...], preferred_element_type=jnp.float32)
         + jnp.dot(agg_d, V3[...], preferred_element_type=jnp.float32)
         + jnp.dot(agg_t, V4[...], preferred_element_type=jnp.float32)
         + b_ud[...])
    out[...] = _layer_norm(h + _elu(z), g_d[...], be_d[...])


def _device_update(hd, so, co, sd, cd, st, ct, W_ud, b_ud, g_d, be_d):
    V1, V2, V3, V4 = (W_ud[:H], W_ud[H:2 * H],
                      W_ud[2 * H:3 * H], W_ud[3 * H:])
    grid = 10
    bm = N_DEVICE // grid

    def agg_spec():
        return pl.BlockSpec((NC, bm, H), lambda i: (0, i, 0))

    def cnt_spec():
        return pl.BlockSpec((NC, bm, CL), lambda i: (0, i, 0))

    return pl.pallas_call(
        _device_body,
        grid=(grid,),
        in_specs=[
            pl.BlockSpec((bm, H), lambda i: (i, 0)),
            agg_spec(), cnt_spec(), agg_spec(), cnt_spec(),
            agg_spec(), cnt_spec(),
            pl.BlockSpec((H, H), lambda i: (0, 0)),
            pl.BlockSpec((H, H), lambda i: (0, 0)),
            pl.BlockSpec((H, H), lambda i: (0, 0)),
            pl.BlockSpec((H, H), lambda i: (0, 0)),
            pl.BlockSpec((H,), lambda i: (0,)),
            pl.BlockSpec((H,), lambda i: (0,)),
            pl.BlockSpec((H,), lambda i: (0,)),
        ],
        out_specs=pl.BlockSpec((bm, H), lambda i: (i, 0)),
        out_shape=jax.ShapeDtypeStruct((N_DEVICE, H), jnp.float32),
    )(hd, so, co, sd, cd, st, ct, V1, V2, V3, V4, b_ud, g_d, be_d)


def kernel(x_order, x_device, x_type, src_d2o, dst_d2o, src_t2o, dst_t2o,
           src_o2d, dst_o2d, src_d2d, dst_d2d, src_t2d, dst_t2d,
           W_po, b_po, W_pd, b_pd, W_pt, b_pt, W_uo, b_uo, W_ud, b_ud,
           g_o, be_o, g_d, be_d):
    ho, hd, ht = _project(x_order, x_device, x_type,
                          W_po, b_po, W_pd, b_pd, W_pt, b_pt)

    e_d2o = _pad_edges(src_d2o, dst_d2o, N_ORDER)
    e_t2o = _pad_edges(src_t2o, dst_t2o, N_ORDER)
    e_o2d = _pad_edges(src_o2d, dst_o2d, N_DEVICE)
    e_d2d = _pad_edges(src_d2d, dst_d2d, N_DEVICE)
    e_t2d = _pad_edges(src_t2d, dst_t2d, N_DEVICE)

    hd2 = jnp.stack([hd[:, :HW], hd[:, HW:]])
    ht2 = jnp.stack([ht[:, :HW], ht[:, HW:]])
    s_d2o, s_t2o = _order_sc(hd2, ht2, e_d2o, e_t2o)
    (s_o2d, c_o2d, s_d2d, c_d2d, s_t2d, c_t2d,
     c_d2o, c_t2o) = _device_sc(ho, hd, ht, e_o2d, e_d2d, e_t2d,
                                e_d2o[1], e_t2o[1])

    ho_new = _order_update(ho, s_d2o, c_d2o, s_t2o, c_t2o,
                           W_uo, b_uo, g_o, be_o)
    hd_new = _device_update(hd, s_o2d, c_o2d, s_d2d, c_d2d, s_t2d, c_t2d,
                            W_ud, b_ud, g_d, be_d)
    return (ho_new, hd_new)


# speedup vs baseline: 1.6514x; 1.0344x over previous
import functools

import jax
import jax.numpy as jnp
from jax import lax
from jax.experimental import pallas as pl
from jax.experimental.pallas import tpu as pltpu
from jax.experimental.pallas import tpu_sc as plsc

N_ORDER = 50000
N_DEVICE = 10000
N_TYPE = 64
H = 64

NC = 2
NS = 16
CH = 128
CL = 8
R = 4

HALF_O = N_ORDER // NC
ACC_O = 25088
ACC_D = 10112


def _elu(x):
    return jnp.where(x > 0, x, jnp.exp(jnp.minimum(x, 0.0)) - 1.0)


def _proj_body(xo, xd, xt, W_po, b_po, W_pd, b_pd, W_pt, b_pt,
               ho, hd, ht):
    ho[...] = _elu(jnp.dot(xo[...], W_po[...],
                           preferred_element_type=jnp.float32) + b_po[...])
    hd[...] = _elu(jnp.dot(xd[...], W_pd[...],
                           preferred_element_type=jnp.float32) + b_pd[...])
    ht[...] = _elu(xt[...] * W_pt[...] + b_pt[...])


def _project(xo, xd, xt, W_po, b_po, W_pd, b_pd, W_pt, b_pt):
    grid = 10
    bo, bd = N_ORDER // grid, N_DEVICE // grid
    return pl.pallas_call(
        _proj_body,
        grid=(grid,),
        in_specs=[
            pl.BlockSpec((bo, 5), lambda i: (i, 0)),
            pl.BlockSpec((bd, 6), lambda i: (i, 0)),
            pl.BlockSpec((N_TYPE, 1), lambda i: (0, 0)),
            pl.BlockSpec((5, H), lambda i: (0, 0)),
            pl.BlockSpec((H,), lambda i: (0,)),
            pl.BlockSpec((6, H), lambda i: (0, 0)),
            pl.BlockSpec((H,), lambda i: (0,)),
            pl.BlockSpec((1, H), lambda i: (0, 0)),
            pl.BlockSpec((H,), lambda i: (0,)),
        ],
        out_specs=[
            pl.BlockSpec((bo, H), lambda i: (i, 0)),
            pl.BlockSpec((bd, H), lambda i: (i, 0)),
            pl.BlockSpec((N_TYPE, H), lambda i: (0, 0)),
        ],
        out_shape=[
            jax.ShapeDtypeStruct((N_ORDER, H), jnp.float32),
            jax.ShapeDtypeStruct((N_DEVICE, H), jnp.float32),
            jax.ShapeDtypeStruct((N_TYPE, H), jnp.float32),
        ],
    )(xo, xd, xt, W_po, b_po, W_pd, b_pd, W_pt, b_pt)


ACC_CNT_O = 50176


def _pad_edges(src, dst, dst_fill):
    E = src.shape[0]
    mult = NC * NS * R * CH
    Ep = ((E + mult - 1) // mult) * mult
    if Ep != E:
        pad = Ep - E
        src = jnp.concatenate([src, jnp.zeros((pad,), jnp.int32)])
        dst = jnp.concatenate([dst, jnp.full((pad,), dst_fill, jnp.int32)])
    return src, dst


def _scan_pipe(rr, cpt, chunk0, base, remap, tab, s2, d2,
               sets, gsems, ssems, isems, acc, cnt, ones_v):
    n_super = cpt // rr
    W = rr * CH
    assert n_super >= 2 and (n_super - 2) % 3 == 0 or n_super == 2, n_super
    T = (n_super - 2) // 3

    def issue_idx(p, sup):
        e0 = (chunk0 + sup * rr) * CH
        if tab is not None:
            pltpu.async_copy(s2.at[pl.ds(e0, W)], sets[p][0], isems[p])
        pltpu.async_copy(d2.at[pl.ds(e0, W)], sets[p][1], isems[p])

    def wait_idx(p):
        if tab is not None:
            pltpu.make_async_copy(s2.at[pl.ds(0, W)], sets[p][0],
                                  isems[p]).wait()
        pltpu.make_async_copy(d2.at[pl.ds(0, W)], sets[p][1],
                              isems[p]).wait()

    def drain_scatter(p):
        if tab is not None:
            pltpu.make_async_copy(sets[p][2], acc.at[sets[p][1]],
                                  ssems[p]).wait()
        if cnt is not None:
            pltpu.make_async_copy(ones_v, cnt.at[sets[p][1]],
                                  ssems[p]).wait()

    def stage(p):
        idx_s, idx_d, rbuf = sets[p]
        wait_idx(p)
        gd = None
        if tab is not None:
            gd = pltpu.async_copy(tab.at[idx_s], rbuf, gsems[p])
        if remap:
            for i in range(W // 16):
                d = idx_d[pl.ds(i * 16, 16)] - base
                ok = (d >= 0) & (d < HALF_O)
                idx_d[pl.ds(i * 16, 16)] = jnp.where(ok, d, HALF_O)
        if gd is not None:
            gd.wait()
            pltpu.async_copy(rbuf, acc.at[idx_d], ssems[p], add=True)
        if cnt is not None:
            pltpu.async_copy(ones_v, cnt.at[idx_d], ssems[p], add=True)

    issue_idx(0, 0)
    issue_idx(1, 1)
    stage(0)
    if n_super > 2:
        issue_idx(2, 2)
    stage(1)
    drain_scatter(0)
    if n_super > 3:
        issue_idx(0, 3)

    if T > 0:
        def body(t, carry):
            for k in range(3):
                g = 2 + 3 * t + k
                p = (2 + k) % 3
                q = (p + 2) % 3
                stage(p)
                drain_scatter(q)
                issue_idx(q, jnp.minimum(g + 2, n_super - 1))
            return carry

        lax.fori_loop(0, T, body, 0)
        wait_idx((n_super - 3) % 3)
        wait_idx((n_super - 2) % 3)
    drain_scatter((n_super - 1) % 3)


HW = H // NC
ACC_FULL_O = 50176


def _order_sc(hd2, ht2, e_d2o, e_t2o):
    rr = 1
    cpt = [e[0].shape[0] // CH // NS for e in (e_d2o, e_t2o)]
    rpt = ACC_FULL_O // NS
    z32 = jnp.zeros((rpt, HW), jnp.float32)

    def body(hd_t, sd2o, dd2o, ht_t, st2o, dt2o, z32_t,
             out_d2o, out_t2o, *scr):
        sets = [scr[3 * j:3 * j + 3] for j in range(3)]
        gsems, ssems, isems = scr[9:12], scr[12:15], scr[15:18]
        c = lax.axis_index("c")
        s = lax.axis_index("s")
        phases = [(hd_t.at[c], sd2o, dd2o, cpt[0], out_d2o),
                  (ht_t.at[c], st2o, dt2o, cpt[1], out_t2o)]
        acc = scr[18]
        pltpu.sync_copy(z32_t.at[pl.ds(0, rpt)],
                        acc.at[pl.ds(s * rpt, rpt)])
        plsc.subcore_barrier()
        for pi, (tab, s2, d2, cp, out) in enumerate(phases):
            _scan_pipe(rr, cp, s * cp, 0, False, tab, s2, d2,
                       sets, gsems, ssems, isems, acc, None, None)
            plsc.subcore_barrier()
            pltpu.sync_copy(acc.at[pl.ds(s * rpt, rpt)],
                            out.at[c, pl.ds(s * rpt, rpt)])
            if pi + 1 < len(phases):
                pltpu.sync_copy(z32_t.at[pl.ds(0, rpt)],
                                acc.at[pl.ds(s * rpt, rpt)])
                plsc.subcore_barrier()

    mesh = plsc.VectorSubcoreMesh(core_axis_name="c", subcore_axis_name="s")
    W = rr * CH
    f = pl.kernel(
        body,
        out_type=[jax.ShapeDtypeStruct((NC, ACC_FULL_O, HW),
                                       jnp.float32)] * 2,
        mesh=mesh,
        scratch_types=(
            [pltpu.VMEM((W,), jnp.int32),
             pltpu.VMEM((W,), jnp.int32),
             pltpu.VMEM((W, HW), jnp.float32)] * 3
            + [pltpu.SemaphoreType.DMA] * 9
            + [pltpu.VMEM_SHARED((ACC_FULL_O, HW), jnp.float32)]),
        compiler_params=pltpu.CompilerParams(use_tc_tiling_on_sc=False),
    )
    return f(hd2, e_d2o[0], e_d2o[1], ht2, e_t2o[0], e_t2o[1], z32)


def _counts_sc(dd2o, dt2o):
    rr = 2
    cpt_co = [d.shape[0] // CH // (NC * NS) for d in (dd2o, dt2o)]
    rpto = ACC_CNT_O // NS
    W = rr * CH
    z8 = jnp.zeros((rpto, CL), jnp.float32)
    ones_hbm = jnp.ones((W, CL), jnp.float32)

    def body(dd2o_t, dt2o_t, z8_t, ones_t, o_cd2o, o_ct2o, *scr):
        sets = [(None, scr[j], None) for j in range(3)]
        ones_v = scr[3]
        ssems, isems = scr[4:7], scr[7:10]
        gsems = (None, None, None)
        cnt_o = scr[10]
        c = lax.axis_index("c")
        s = lax.axis_index("s")
        tile = s * NC + c
        pltpu.sync_copy(z8_t.at[pl.ds(0, rpto)],
                        cnt_o.at[pl.ds(s * rpto, rpto)])
        pltpu.sync_copy(ones_t, ones_v)
        plsc.subcore_barrier()
        cphases = [(dd2o_t, cpt_co[0], o_cd2o), (dt2o_t, cpt_co[1], o_ct2o)]
        for pi, (d2, cp, out_c) in enumerate(cphases):
            _scan_pipe(rr, cp, tile * cp, 0, False, None, None, d2,
                       sets, gsems, ssems, isems, None, cnt_o, ones_v)
            plsc.subcore_barrier()
            pltpu.sync_copy(cnt_o.at[pl.ds(s * rpto, rpto)],
                            out_c.at[c, pl.ds(s * rpto, rpto)])
            if pi + 1 < len(cphases):
                pltpu.sync_copy(z8_t.at[pl.ds(0, rpto)],
                                cnt_o.at[pl.ds(s * rpto, rpto)])
                plsc.subcore_barrier()

    mesh = plsc.VectorSubcoreMesh(core_axis_name="c", subcore_axis_name="s")
    f = pl.kernel(
        body,
        out_type=[jax.ShapeDtypeStruct((NC, ACC_CNT_O, CL),
                                       jnp.float32)] * 2,
        mesh=mesh,
        scratch_types=(
            [pltpu.VMEM((W,), jnp.int32)] * 3
            + [pltpu.VMEM((W, CL), jnp.float32)]
            + [pltpu.SemaphoreType.DMA] * 6
            + [pltpu.VMEM_SHARED((ACC_CNT_O, CL), jnp.float32)]),
        compiler_params=pltpu.CompilerParams(use_tc_tiling_on_sc=False),
    )
    return f(dd2o, dt2o, z8, ones_hbm)


def _device_sc(ho, hd, ht, e_o2d, e_d2d, e_t2d):
    rr = 2
    cpt = [e[0].shape[0] // CH // (NC * NS) for e in (e_o2d, e_d2d, e_t2d)]
    rptd = ACC_D // NS
    W = rr * CH
    z64 = jnp.zeros((rptd, H), jnp.float32)
    z8 = jnp.zeros((rptd, CL), jnp.float32)
    ones_hbm = jnp.ones((W, CL), jnp.float32)

    def body(ho_t, so2d, do2d, hd_t, sd2d, dd2d, ht_t, st2d, dt2d,
             z64_t, z8_t, ones_t,
             o_so2d, o_co2d, o_sd2d, o_cd2d, o_st2d, o_ct2d, *scr):
        sets = [scr[3 * j:3 * j + 3] for j in range(3)]
        ones_v = scr[9]
        gsems, ssems, isems = scr[10:13], scr[13:16], scr[16:19]
        acc, cnt = scr[19], scr[20]
        c = lax.axis_index("c")
        s = lax.axis_index("s")
        tile = s * NC + c
        pltpu.sync_copy(z64_t.at[pl.ds(0, rptd)],
                        acc.at[pl.ds(s * rptd, rptd)])
        pltpu.sync_copy(z8_t.at[pl.ds(0, rptd)],
                        cnt.at[pl.ds(s * rptd, rptd)])
        pltpu.sync_copy(ones_t, ones_v)
        plsc.subcore_barrier()

        phases = [(ho_t, so2d, do2d, cpt[0], o_so2d, o_co2d),
                  (hd_t, sd2d, dd2d, cpt[1], o_sd2d, o_cd2d),
                  (ht_t, st2d, dt2d, cpt[2], o_st2d, o_ct2d)]
        for pi, (tab, s2, d2, cp, out_s, out_c) in enumerate(phases):
            _scan_pipe(rr, cp, tile * cp, 0, False, tab, s2, d2,
                       sets, gsems, ssems, isems, acc, cnt, ones_v)
            plsc.subcore_barrier()
            pltpu.sync_copy(acc.at[pl.ds(s * rptd, rptd)],
                            out_s.at[c, pl.ds(s * rptd, rptd)])
            pltpu.sync_copy(cnt.at[pl.ds(s * rptd, rptd)],
                            out_c.at[c, pl.ds(s * rptd, rptd)])
            if pi + 1 < len(phases):
                pltpu.sync_copy(z64_t.at[pl.ds(0, rptd)],
                                acc.at[pl.ds(s * rptd, rptd)])
                pltpu.sync_copy(z8_t.at[pl.ds(0, rptd)],
                                cnt.at[pl.ds(s * rptd, rptd)])
                plsc.subcore_barrier()

    mesh = plsc.VectorSubcoreMesh(core_axis_name="c", subcore_axis_name="s")
    f = pl.kernel(
        body,
        out_type=[jax.ShapeDtypeStruct((NC, ACC_D, H), jnp.float32),
                  jax.ShapeDtypeStruct((NC, ACC_D, CL), jnp.float32)] * 3,
        mesh=mesh,
        scratch_types=(
            [pltpu.VMEM((W,), jnp.int32),
             pltpu.VMEM((W,), jnp.int32),
             pltpu.VMEM((W, H), jnp.float32)] * 3
            + [pltpu.VMEM((W, CL), jnp.float32)]
            + [pltpu.SemaphoreType.DMA] * 9
            + [pltpu.VMEM_SHARED((ACC_D, H), jnp.float32),
               pltpu.VMEM_SHARED((ACC_D, CL), jnp.float32)]),
        compiler_params=pltpu.CompilerParams(use_tc_tiling_on_sc=False),
    )
    return f(ho, e_o2d[0], e_o2d[1], hd, e_d2d[0], e_d2d[1],
             ht, e_t2d[0], e_t2d[1], z64, z8, ones_hbm)


def _layer_norm(x, g, b, eps=1e-5):
    mu = jnp.mean(x, axis=-1, keepdims=True)
    xc = x - mu
    var = jnp.mean(xc * xc, axis=-1, keepdims=True)
    return xc * lax.rsqrt(var + eps) * g + b


def _mean2(sum_ref, cnt_ref):
    s = sum_ref[0] + sum_ref[1]
    n = cnt_ref[0][:, 0:1] + cnt_ref[1][:, 0:1]
    return s / jnp.maximum(n, 1.0)


def _order_body(ho, sum_d, cnt_d, sum_t, cnt_t,
                W1, W2, W3, b_uo, g_o, be_o, out):
    n_d = cnt_d[0][:, 0:1] + cnt_d[1][:, 0:1]
    n_t = cnt_t[0][:, 0:1] + cnt_t[1][:, 0:1]
    agg_d = jnp.concatenate([sum_d[0], sum_d[1]],
                            axis=-1) / jnp.maximum(n_d, 1.0)
    agg_t = jnp.concatenate([sum_t[0], sum_t[1]],
                            axis=-1) / jnp.maximum(n_t, 1.0)
    h = ho[...]
    z = (jnp.dot(h, W1[...], preferred_element_type=jnp.float32)
         + jnp.dot(agg_d, W2[...], preferred_element_type=jnp.float32)
         + jnp.dot(agg_t, W3[...], preferred_element_type=jnp.float32)
         + b_uo[...])
    out[...] = _layer_norm(h + _elu(z), g_o[...], be_o[...])


def _order_update(ho, sums_d, cnts_d, sums_t, cnts_t, W_uo, b_uo, g_o, be_o):
    W1, W2, W3 = W_uo[:H], W_uo[H:2 * H], W_uo[2 * H:]
    grid = 50
    bm = N_ORDER // grid

    def agg_spec():
        return pl.BlockSpec((NC, bm, HW), lambda i: (0, i, 0))

    def cnt_spec():
        return pl.BlockSpec((NC, bm, CL), lambda i: (0, i, 0))

    return pl.pallas_call(
        _order_body,
        grid=(grid,),
        in_specs=[
            pl.BlockSpec((bm, H), lambda i: (i, 0)),
            agg_spec(), cnt_spec(), agg_spec(), cnt_spec(),
            pl.BlockSpec((H, H), lambda i: (0, 0)),
            pl.BlockSpec((H, H), lambda i: (0, 0)),
            pl.BlockSpec((H, H), lambda i: (0, 0)),
            pl.BlockSpec((H,), lambda i: (0,)),
            pl.BlockSpec((H,), lambda i: (0,)),
            pl.BlockSpec((H,), lambda i: (0,)),
        ],
        out_specs=pl.BlockSpec((bm, H), lambda i: (i, 0)),
        out_shape=jax.ShapeDtypeStruct((N_ORDER, H), jnp.float32),
    )(ho, sums_d, cnts_d, sums_t, cnts_t, W1, W2, W3, b_uo, g_o, be_o)


def _device_body(hd, sum_o, cnt_o, sum_d, cnt_d, sum_t, cnt_t,
                 V1, V2, V3, V4, b_ud, g_d, be_d, out):
    agg_o = _mean2(sum_o, cnt_o)
    agg_d = _mean2(sum_d, cnt_d)
    agg_t = _mean2(sum_t, cnt_t)
    h = hd[...]
    z = (jnp.dot(h, V1[...], preferred_element_type=jnp.float32)
         + jnp.dot(agg_o, V2[...], preferred_element_type=jnp.float32)
         + jnp.dot(agg_d, V3[...], preferred_element_type=jnp.float32)
         + jnp.dot(agg_t, V4[...], preferred_element_type=jnp.float32)
         + b_ud[...])
    out[...] = _layer_norm(h + _elu(z), g_d[...], be_d[...])


def _device_update(hd, so, co, sd, cd, st, ct, W_ud, b_ud, g_d, be_d):
    V1, V2, V3, V4 = (W_ud[:H], W_ud[H:2 * H],
                      W_ud[2 * H:3 * H], W_ud[3 * H:])
    grid = 10
    bm = N_DEVICE // grid

    def agg_spec():
        return pl.BlockSpec((NC, bm, H), lambda i: (0, i, 0))

    def cnt_spec():
        return pl.BlockSpec((NC, bm, CL), lambda i: (0, i, 0))

    return pl.pallas_call(
        _device_body,
        grid=(grid,),
        in_specs=[
            pl.BlockSpec((bm, H), lambda i: (i, 0)),
            agg_spec(), cnt_spec(), agg_spec(), cnt_spec(),
            agg_spec(), cnt_spec(),
            pl.BlockSpec((H, H), lambda i: (0, 0)),
            pl.BlockSpec((H, H), lambda i: (0, 0)),
            pl.BlockSpec((H, H), lambda i: (0, 0)),
            pl.BlockSpec((H, H), lambda i: (0, 0)),
            pl.BlockSpec((H,), lambda i: (0,)),
            pl.BlockSpec((H,), lambda i: (0,)),
            pl.BlockSpec((H,), lambda i: (0,)),
        ],
        out_specs=pl.BlockSpec((bm, H), lambda i: (i, 0)),
        out_shape=jax.ShapeDtypeStruct((N_DEVICE, H), jnp.float32),
    )(hd, so, co, sd, cd, st, ct, V1, V2, V3, V4, b_ud, g_d, be_d)


def kernel(x_order, x_device, x_type, src_d2o, dst_d2o, src_t2o, dst_t2o,
           src_o2d, dst_o2d, src_d2d, dst_d2d, src_t2d, dst_t2d,
           W_po, b_po, W_pd, b_pd, W_pt, b_pt, W_uo, b_uo, W_ud, b_ud,
           g_o, be_o, g_d, be_d):
    e_d2o = _pad_edges(src_d2o, dst_d2o, N_ORDER)
    e_t2o = _pad_edges(src_t2o, dst_t2o, N_ORDER)
    e_o2d = _pad_edges(src_o2d, dst_o2d, N_DEVICE)
    e_d2d = _pad_edges(src_d2d, dst_d2d, N_DEVICE)
    e_t2d = _pad_edges(src_t2d, dst_t2d, N_DEVICE)

    c_d2o, c_t2o = _counts_sc(e_d2o[1], e_t2o[1])
    ho, hd, ht = _project(x_order, x_device, x_type,
                          W_po, b_po, W_pd, b_pd, W_pt, b_pt)

    hd2 = jnp.stack([hd[:, :HW], hd[:, HW:]])
    ht2 = jnp.stack([ht[:, :HW], ht[:, HW:]])
    s_d2o, s_t2o = _order_sc(hd2, ht2, e_d2o, e_t2o)
    (s_o2d, c_o2d, s_d2d, c_d2d,
     s_t2d, c_t2d) = _device_sc(ho, hd, ht, e_o2d, e_d2d, e_t2d)

    ho_new = _order_update(ho, s_d2o, c_d2o, s_t2o, c_t2o,
                           W_uo, b_uo, g_o, be_o)
    hd_new = _device_update(hd, s_o2d, c_o2d, s_d2d, c_d2d, s_t2d, c_t2d,
                            W_ud, b_ud, g_d, be_d)
    return (ho_new, hd_new)


# speedup vs baseline: 1.8054x; 1.0933x over previous
import functools

import jax
import jax.numpy as jnp
from jax import lax
from jax.experimental import pallas as pl
from jax.experimental.pallas import tpu as pltpu
from jax.experimental.pallas import tpu_sc as plsc

N_ORDER = 50000
N_DEVICE = 10000
N_TYPE = 64
H = 64

NC = 2
NS = 16
CH = 128
CL = 8
R = 4

HALF_O = N_ORDER // NC
ACC_O = 25088
ACC_D = 10112


def _elu(x):
    return jnp.where(x > 0, x, jnp.exp(jnp.minimum(x, 0.0)) - 1.0)


def _proj_body(xo, xd, xt, W_po, b_po, W_pd, b_pd, W_pt, b_pt,
               ho, hd, ht):
    ho[...] = _elu(jnp.dot(xo[...], W_po[...],
                           preferred_element_type=jnp.float32) + b_po[...])
    hd[...] = _elu(jnp.dot(xd[...], W_pd[...],
                           preferred_element_type=jnp.float32) + b_pd[...])
    ht[...] = _elu(xt[...] * W_pt[...] + b_pt[...])


def _project(xo, xd, xt, W_po, b_po, W_pd, b_pd, W_pt, b_pt):
    grid = 10
    bo, bd = N_ORDER // grid, N_DEVICE // grid
    return pl.pallas_call(
        _proj_body,
        grid=(grid,),
        in_specs=[
            pl.BlockSpec((bo, 5), lambda i: (i, 0)),
            pl.BlockSpec((bd, 6), lambda i: (i, 0)),
            pl.BlockSpec((N_TYPE, 1), lambda i: (0, 0)),
            pl.BlockSpec((5, H), lambda i: (0, 0)),
            pl.BlockSpec((H,), lambda i: (0,)),
            pl.BlockSpec((6, H), lambda i: (0, 0)),
            pl.BlockSpec((H,), lambda i: (0,)),
            pl.BlockSpec((1, H), lambda i: (0, 0)),
            pl.BlockSpec((H,), lambda i: (0,)),
        ],
        out_specs=[
            pl.BlockSpec((bo, H), lambda i: (i, 0)),
            pl.BlockSpec((bd, H), lambda i: (i, 0)),
            pl.BlockSpec((N_TYPE, H), lambda i: (0, 0)),
        ],
        out_shape=[
            jax.ShapeDtypeStruct((N_ORDER, H), jnp.float32),
            jax.ShapeDtypeStruct((N_DEVICE, H), jnp.float32),
            jax.ShapeDtypeStruct((N_TYPE, H), jnp.float32),
        ],
    )(xo, xd, xt, W_po, b_po, W_pd, b_pd, W_pt, b_pt)


ACC_CNT_O = 50176


def _pad_edges(src, dst, dst_fill):
    E = src.shape[0]
    mult = NC * NS * R * CH
    Ep = ((E + mult - 1) // mult) * mult
    if Ep != E:
        pad = Ep - E
        src = jnp.concatenate([src, jnp.zeros((pad,), jnp.int32)])
        dst = jnp.concatenate([dst, jnp.full((pad,), dst_fill, jnp.int32)])
    return src, dst


def _scan_pipe(rr, cpt, chunk0, base, remap, tab, s2, d2,
               sets, gsems, ssems, isems, acc, cnt, ones_v):
    n_super = cpt // rr
    W = rr * CH
    assert n_super == 2 or (n_super > 2 and (n_super - 2) % 3 in (0, 2)), \
        n_super
    T = (n_super - 2) // 3
    r = 0 if n_super == 2 else (n_super - 2) % 3

    def issue_idx(p, sup):
        e0 = (chunk0 + sup * rr) * CH
        if tab is not None:
            pltpu.async_copy(s2.at[pl.ds(e0, W)], sets[p][0], isems[p])
        pltpu.async_copy(d2.at[pl.ds(e0, W)], sets[p][1], isems[p])

    def wait_idx(p):
        if tab is not None:
            pltpu.make_async_copy(s2.at[pl.ds(0, W)], sets[p][0],
                                  isems[p]).wait()
        pltpu.make_async_copy(d2.at[pl.ds(0, W)], sets[p][1],
                              isems[p]).wait()

    def drain_scatter(p):
        if tab is not None:
            pltpu.make_async_copy(sets[p][2], acc.at[sets[p][1]],
                                  ssems[p]).wait()
        if cnt is not None:
            pltpu.make_async_copy(ones_v, cnt.at[sets[p][1]],
                                  ssems[p]).wait()

    def stage(p):
        idx_s, idx_d, rbuf = sets[p]
        wait_idx(p)
        gd = None
        if tab is not None:
            gd = pltpu.async_copy(tab.at[idx_s], rbuf, gsems[p])
        if remap:
            for i in range(W // 16):
                d = idx_d[pl.ds(i * 16, 16)] - base
                ok = (d >= 0) & (d < HALF_O)
                idx_d[pl.ds(i * 16, 16)] = jnp.where(ok, d, HALF_O)
        if gd is not None:
            gd.wait()
            pltpu.async_copy(rbuf, acc.at[idx_d], ssems[p], add=True)
        if cnt is not None:
            pltpu.async_copy(ones_v, cnt.at[idx_d], ssems[p], add=True)

    issue_idx(0, 0)
    issue_idx(1, 1)
    stage(0)
    if n_super > 2:
        issue_idx(2, 2)
    stage(1)
    drain_scatter(0)
    if n_super > 3:
        issue_idx(0, 3)

    if T > 0:
        def body(t, carry):
            for k in range(3):
                g = 2 + 3 * t + k
                p = (2 + k) % 3
                q = (p + 2) % 3
                stage(p)
                drain_scatter(q)
                issue_idx(q, jnp.minimum(g + 2, n_super - 1))
            return carry

        lax.fori_loop(0, T, body, 0)
        if r == 0:
            wait_idx((n_super - 3) % 3)
            wait_idx((n_super - 2) % 3)
        for j in range(r):
            g = n_super - r + j
            stage(g % 3)
            drain_scatter((g - 1) % 3)
    drain_scatter((n_super - 1) % 3)


HW = H // NC
ACC_FULL_O = 50176


def _order_sc(hd2, ht2, e_d2o, e_t2o):
    rr = 2
    cpt = [e[0].shape[0] // CH // NS for e in (e_d2o, e_t2o)]
    rpt = ACC_FULL_O // NS
    z32 = jnp.zeros((rpt, HW), jnp.float32)

    def body(hd_t, sd2o, dd2o, ht_t, st2o, dt2o, z32_t,
             out_d2o, out_t2o, *scr):
        sets = [scr[3 * j:3 * j + 3] for j in range(3)]
        gsems, ssems, isems = scr[9:12], scr[12:15], scr[15:18]
        c = lax.axis_index("c")
        s = lax.axis_index("s")
        phases = [(hd_t.at[c], sd2o, dd2o, cpt[0], out_d2o),
                  (ht_t.at[c], st2o, dt2o, cpt[1], out_t2o)]
        acc = scr[18]
        pltpu.sync_copy(z32_t.at[pl.ds(0, rpt)],
                        acc.at[pl.ds(s * rpt, rpt)])
        plsc.subcore_barrier()
        for pi, (tab, s2, d2, cp, out) in enumerate(phases):
            _scan_pipe(rr, cp, s * cp, 0, False, tab, s2, d2,
                       sets, gsems, ssems, isems, acc, None, None)
            plsc.subcore_barrier()
            pltpu.sync_copy(acc.at[pl.ds(s * rpt, rpt)],
                            out.at[c, pl.ds(s * rpt, rpt)])
            if pi + 1 < len(phases):
                pltpu.sync_copy(z32_t.at[pl.ds(0, rpt)],
                                acc.at[pl.ds(s * rpt, rpt)])
                plsc.subcore_barrier()

    mesh = plsc.VectorSubcoreMesh(core_axis_name="c", subcore_axis_name="s")
    W = rr * CH
    f = pl.kernel(
        body,
        out_type=[jax.ShapeDtypeStruct((NC, ACC_FULL_O, HW),
                                       jnp.float32)] * 2,
        mesh=mesh,
        scratch_types=(
            [pltpu.VMEM((W,), jnp.int32),
             pltpu.VMEM((W,), jnp.int32),
             pltpu.VMEM((W, HW), jnp.float32)] * 3
            + [pltpu.SemaphoreType.DMA] * 9
            + [pltpu.VMEM_SHARED((ACC_FULL_O, HW), jnp.float32)]),
        compiler_params=pltpu.CompilerParams(use_tc_tiling_on_sc=False),
    )
    return f(hd2, e_d2o[0], e_d2o[1], ht2, e_t2o[0], e_t2o[1], z32)


def _counts_sc(dd2o, dt2o):
    rr = 2
    cpt_co = [d.shape[0] // CH // (NC * NS) for d in (dd2o, dt2o)]
    rpto = ACC_CNT_O // NS
    W = rr * CH
    z8 = jnp.zeros((rpto, CL), jnp.float32)
    ones_hbm = jnp.ones((W, CL), jnp.float32)

    def body(dd2o_t, dt2o_t, z8_t, ones_t, o_cd2o, o_ct2o, *scr):
        sets = [(None, scr[j], None) for j in range(3)]
        ones_v = scr[3]
        ssems, isems = scr[4:7], scr[7:10]
        gsems = (None, None, None)
        cnt_o = scr[10]
        c = lax.axis_index("c")
        s = lax.axis_index("s")
        tile = s * NC + c
        pltpu.sync_copy(z8_t.at[pl.ds(0, rpto)],
                        cnt_o.at[pl.ds(s * rpto, rpto)])
        pltpu.sync_copy(ones_t, ones_v)
        plsc.subcore_barrier()
        cphases = [(dd2o_t, cpt_co[0], o_cd2o), (dt2o_t, cpt_co[1], o_ct2o)]
        for pi, (d2, cp, out_c) in enumerate(cphases):
            _scan_pipe(rr, cp, tile * cp, 0, False, None, None, d2,
                       sets, gsems, ssems, isems, None, cnt_o, ones_v)
            plsc.subcore_barrier()
            pltpu.sync_copy(cnt_o.at[pl.ds(s * rpto, rpto)],
                            out_c.at[c, pl.ds(s * rpto, rpto)])
            if pi + 1 < len(cphases):
                pltpu.sync_copy(z8_t.at[pl.ds(0, rpto)],
                                cnt_o.at[pl.ds(s * rpto, rpto)])
                plsc.subcore_barrier()

    mesh = plsc.VectorSubcoreMesh(core_axis_name="c", subcore_axis_name="s")
    f = pl.kernel(
        body,
        out_type=[jax.ShapeDtypeStruct((NC, ACC_CNT_O, CL),
                                       jnp.float32)] * 2,
        mesh=mesh,
        scratch_types=(
            [pltpu.VMEM((W,), jnp.int32)] * 3
            + [pltpu.VMEM((W, CL), jnp.float32)]
            + [pltpu.SemaphoreType.DMA] * 6
            + [pltpu.VMEM_SHARED((ACC_CNT_O, CL), jnp.float32)]),
        compiler_params=pltpu.CompilerParams(use_tc_tiling_on_sc=False),
    )
    return f(dd2o, dt2o, z8, ones_hbm)


def _device_sc(ho, hd, ht, e_o2d, e_d2d, e_t2d):
    rr = 2
    cpt = [e[0].shape[0] // CH // (NC * NS) for e in (e_o2d, e_d2d, e_t2d)]
    rptd = ACC_D // NS
    W = rr * CH
    z64 = jnp.zeros((rptd, H), jnp.float32)
    z8 = jnp.zeros((rptd, CL), jnp.float32)
    ones_hbm = jnp.ones((W, CL), jnp.float32)

    def body(ho_t, so2d, do2d, hd_t, sd2d, dd2d, ht_t, st2d, dt2d,
             z64_t, z8_t, ones_t,
             o_so2d, o_co2d, o_sd2d, o_cd2d, o_st2d, o_ct2d, *scr):
        sets = [scr[3 * j:3 * j + 3] for j in range(3)]
        ones_v = scr[9]
        gsems, ssems, isems = scr[10:13], scr[13:16], scr[16:19]
        acc, cnt = scr[19], scr[20]
        c = lax.axis_index("c")
        s = lax.axis_index("s")
        tile = s * NC + c
        pltpu.sync_copy(z64_t.at[pl.ds(0, rptd)],
                        acc.at[pl.ds(s * rptd, rptd)])
        pltpu.sync_copy(z8_t.at[pl.ds(0, rptd)],
                        cnt.at[pl.ds(s * rptd, rptd)])
        pltpu.sync_copy(ones_t, ones_v)
        plsc.subcore_barrier()

        phases = [(ho_t, so2d, do2d, cpt[0], o_so2d, o_co2d),
                  (hd_t, sd2d, dd2d, cpt[1], o_sd2d, o_cd2d),
                  (ht_t, st2d, dt2d, cpt[2], o_st2d, o_ct2d)]
        for pi, (tab, s2, d2, cp, out_s, out_c) in enumerate(phases):
            _scan_pipe(rr, cp, tile * cp, 0, False, tab, s2, d2,
                       sets, gsems, ssems, isems, acc, cnt, ones_v)
            plsc.subcore_barrier()
            pltpu.sync_copy(acc.at[pl.ds(s * rptd, rptd)],
                            out_s.at[c, pl.ds(s * rptd, rptd)])
            pltpu.sync_copy(cnt.at[pl.ds(s * rptd, rptd)],
                            out_c.at[c, pl.ds(s * rptd, rptd)])
            if pi + 1 < len(phases):
                pltpu.sync_copy(z64_t.at[pl.ds(0, rptd)],
                                acc.at[pl.ds(s * rptd, rptd)])
                pltpu.sync_copy(z8_t.at[pl.ds(0, rptd)],
                                cnt.at[pl.ds(s * rptd, rptd)])
                plsc.subcore_barrier()

    mesh = plsc.VectorSubcoreMesh(core_axis_name="c", subcore_axis_name="s")
    f = pl.kernel(
        body,
        out_type=[jax.ShapeDtypeStruct((NC, ACC_D, H), jnp.float32),
                  jax.ShapeDtypeStruct((NC, ACC_D, CL), jnp.float32)] * 3,
        mesh=mesh,
        scratch_types=(
            [pltpu.VMEM((W,), jnp.int32),
             pltpu.VMEM((W,), jnp.int32),
             pltpu.VMEM((W, H), jnp.float32)] * 3
            + [pltpu.VMEM((W, CL), jnp.float32)]
            + [pltpu.SemaphoreType.DMA] * 9
            + [pltpu.VMEM_SHARED((ACC_D, H), jnp.float32),
               pltpu.VMEM_SHARED((ACC_D, CL), jnp.float32)]),
        compiler_params=pltpu.CompilerParams(use_tc_tiling_on_sc=False),
    )
    return f(ho, e_o2d[0], e_o2d[1], hd, e_d2d[0], e_d2d[1],
             ht, e_t2d[0], e_t2d[1], z64, z8, ones_hbm)


def _layer_norm(x, g, b, eps=1e-5):
    mu = jnp.mean(x, axis=-1, keepdims=True)
    xc = x - mu
    var = jnp.mean(xc * xc, axis=-1, keepdims=True)
    return xc * lax.rsqrt(var + eps) * g + b


def _mean2(sum_ref, cnt_ref):
    s = sum_ref[0] + sum_ref[1]
    n = cnt_ref[0][:, 0:1] + cnt_ref[1][:, 0:1]
    return s / jnp.maximum(n, 1.0)


def _order_body(ho, sum_d, cnt_d, sum_t, cnt_t,
                W1, W2, W3, b_uo, g_o, be_o, out):
    n_d = cnt_d[0][:, 0:1] + cnt_d[1][:, 0:1]
    n_t = cnt_t[0][:, 0:1] + cnt_t[1][:, 0:1]
    agg_d = jnp.concatenate([sum_d[0], sum_d[1]],
                            axis=-1) / jnp.maximum(n_d, 1.0)
    agg_t = jnp.concatenate([sum_t[0], sum_t[1]],
                            axis=-1) / jnp.maximum(n_t, 1.0)
    h = ho[...]
    z = (jnp.dot(h, W1[...], preferred_element_type=jnp.float32)
         + jnp.dot(agg_d, W2[...], preferred_element_type=jnp.float32)
         + jnp.dot(agg_t, W3[...], preferred_element_type=jnp.float32)
         + b_uo[...])
    out[...] = _layer_norm(h + _elu(z), g_o[...], be_o[...])


def _order_update(ho, sums_d, cnts_d, sums_t, cnts_t, W_uo, b_uo, g_o, be_o):
    W1, W2, W3 = W_uo[:H], W_uo[H:2 * H], W_uo[2 * H:]
    grid = 50
    bm = N_ORDER // grid

    def agg_spec():
        return pl.BlockSpec((NC, bm, HW), lambda i: (0, i, 0))

    def cnt_spec():
        return pl.BlockSpec((NC, bm, CL), lambda i: (0, i, 0))

    return pl.pallas_call(
        _order_body,
        grid=(grid,),
        in_specs=[
            pl.BlockSpec((bm, H), lambda i: (i, 0)),
            agg_spec(), cnt_spec(), agg_spec(), cnt_spec(),
            pl.BlockSpec((H, H), lambda i: (0, 0)),
            pl.BlockSpec((H, H), lambda i: (0, 0)),
            pl.BlockSpec((H, H), lambda i: (0, 0)),
            pl.BlockSpec((H,), lambda i: (0,)),
            pl.BlockSpec((H,), lambda i: (0,)),
            pl.BlockSpec((H,), lambda i: (0,)),
        ],
        out_specs=pl.BlockSpec((bm, H), lambda i: (i, 0)),
        out_shape=jax.ShapeDtypeStruct((N_ORDER, H), jnp.float32),
    )(ho, sums_d, cnts_d, sums_t, cnts_t, W1, W2, W3, b_uo, g_o, be_o)


def _device_body(hd, sum_o, cnt_o, sum_d, cnt_d, sum_t, cnt_t,
                 V1, V2, V3, V4, b_ud, g_d, be_d, out):
    agg_o = _mean2(sum_o, cnt_o)
    agg_d = _mean2(sum_d, cnt_d)
    agg_t = _mean2(sum_t, cnt_t)
    h = hd[...]
    z = (jnp.dot(h, V1[...], preferred_element_type=jnp.float32)
         + jnp.dot(agg_o, V2[...], preferred_element_type=jnp.float32)
         + jnp.dot(agg_d, V3[...], preferred_element_type=jnp.float32)
         + jnp.dot(agg_t, V4[...], preferred_element_type=jnp.float32)
         + b_ud[...])
    out[...] = _layer_norm(h + _elu(z), g_d[...], be_d[...])


def _device_update(hd, so, co, sd, cd, st, ct, W_ud, b_ud, g_d, be_d):
    V1, V2, V3, V4 = (W_ud[:H], W_ud[H:2 * H],
                      W_ud[2 * H:3 * H], W_ud[3 * H:])
    grid = 10
    bm = N_DEVICE // grid

    def agg_spec():
        return pl.BlockSpec((NC, bm, H), lambda i: (0, i, 0))

    def cnt_spec():
        return pl.BlockSpec((NC, bm, CL), lambda i: (0, i, 0))

    return pl.pallas_call(
        _device_body,
        grid=(grid,),
        in_specs=[
            pl.BlockSpec((bm, H), lambda i: (i, 0)),
            agg_spec(), cnt_spec(), agg_spec(), cnt_spec(),
            agg_spec(), cnt_spec(),
            pl.BlockSpec((H, H), lambda i: (0, 0)),
            pl.BlockSpec((H, H), lambda i: (0, 0)),
            pl.BlockSpec((H, H), lambda i: (0, 0)),
            pl.BlockSpec((H, H), lambda i: (0, 0)),
            pl.BlockSpec((H,), lambda i: (0,)),
            pl.BlockSpec((H,), lambda i: (0,)),
            pl.BlockSpec((H,), lambda i: (0,)),
        ],
        out_specs=pl.BlockSpec((bm, H), lambda i: (i, 0)),
        out_shape=jax.ShapeDtypeStruct((N_DEVICE, H), jnp.float32),
    )(hd, so, co, sd, cd, st, ct, V1, V2, V3, V4, b_ud, g_d, be_d)


def kernel(x_order, x_device, x_type, src_d2o, dst_d2o, src_t2o, dst_t2o,
           src_o2d, dst_o2d, src_d2d, dst_d2d, src_t2d, dst_t2d,
           W_po, b_po, W_pd, b_pd, W_pt, b_pt, W_uo, b_uo, W_ud, b_ud,
           g_o, be_o, g_d, be_d):
    e_d2o = _pad_edges(src_d2o, dst_d2o, N_ORDER)
    e_t2o = _pad_edges(src_t2o, dst_t2o, N_ORDER)
    e_o2d = _pad_edges(src_o2d, dst_o2d, N_DEVICE)
    e_d2d = _pad_edges(src_d2d, dst_d2d, N_DEVICE)
    e_t2d = _pad_edges(src_t2d, dst_t2d, N_DEVICE)

    c_d2o, c_t2o = _counts_sc(e_d2o[1], e_t2o[1])
    ho, hd, ht = _project(x_order, x_device, x_type,
                          W_po, b_po, W_pd, b_pd, W_pt, b_pt)

    hd2 = jnp.stack([hd[:, :HW], hd[:, HW:]])
    ht2 = jnp.stack([ht[:, :HW], ht[:, HW:]])
    s_d2o, s_t2o = _order_sc(hd2, ht2, e_d2o, e_t2o)
    (s_o2d, c_o2d, s_d2d, c_d2d,
     s_t2d, c_t2d) = _device_sc(ho, hd, ht, e_o2d, e_d2d, e_t2d)

    ho_new = _order_update(ho, s_d2o, c_d2o, s_t2o, c_t2o,
                           W_uo, b_uo, g_o, be_o)
    hd_new = _device_update(hd, s_o2d, c_o2d, s_d2d, c_d2d, s_t2d, c_t2d,
                            W_ud, b_ud, g_d, be_d)
    return (ho_new, hd_new)
